# Initial kernel scaffold; baseline (speedup 1.0000x reference)
import jax
import jax.numpy as jnp
from jax.experimental import pallas as pl


def kernel(x, edge_index, W1, b1, W2, b2, temp):
    raise NotImplementedError("write your pallas kernel here")


# speedup vs baseline: 26.0757x; 26.0757x over previous
import functools
import math

import jax
import jax.numpy as jnp
import numpy as np
from jax import lax
from jax.experimental import pallas as pl
from jax.experimental.pallas import tpu as pltpu
from jax.experimental.pallas import tpu_sc as plsc

K = 10
N = 10000
D = 64
DF = 128
E = 320000

NC, NS = 2, 16
NW = NC * NS
RPT = 640
NPAD = RPT * NS
EPW = 10240
EPAD = EPW * NW
CHUNK = 128
NCHUNK = EPW // CHUNK

ROW_BLK = 1280
GRID = NPAD // ROW_BLK


def _bern_matrix():
    M = np.zeros((K + 1, K + 1), dtype=np.float64)
    for i in range(K + 1):
        for j in range(K + 1):
            g = 0
            for m in range(0, i + 1):
                if 0 <= j - m <= K - i:
                    g += (-1) ** m * math.comb(i, m) * math.comb(K - i, j - m)
            M[j, i] = (math.comb(K, i) / (2 ** K)) * g
    return M


_BERN_M = _bern_matrix()

_SC_MESH = plsc.VectorSubcoreMesh(core_axis_name="c", subcore_axis_name="s")


def _sc_deg_body(row_hbm, ones_hbm, zeros1_hbm, deg_out, idx_v, ones_v, sem,
                 acc):
    cid = lax.axis_index("c")
    sid = lax.axis_index("s")
    wid = sid * NC + cid
    pltpu.sync_copy(zeros1_hbm.at[pl.ds(sid * RPT, RPT)],
                    acc.at[pl.ds(sid * RPT, RPT)])
    pltpu.sync_copy(ones_hbm, ones_v)
    plsc.subcore_barrier()

    ebase = wid * EPW

    def chunk(i, carry):
        base = ebase + i * CHUNK
        pltpu.sync_copy(row_hbm.at[pl.ds(base, CHUNK)], idx_v)
        pltpu.sync_copy(ones_v, acc.at[idx_v], add=True)
        return carry

    lax.fori_loop(0, NCHUNK, chunk, 0)
    plsc.subcore_barrier()
    pltpu.sync_copy(acc.at[pl.ds(sid * RPT, RPT)],
                    deg_out.at[cid, pl.ds(sid * RPT, RPT)])


@functools.partial(
    pl.kernel,
    mesh=_SC_MESH,
    compiler_params=pltpu.CompilerParams(use_tc_tiling_on_sc=False),
    out_type=jax.ShapeDtypeStruct((NC, NPAD), jnp.float32),
    scratch_types=[
        pltpu.VMEM((CHUNK,), jnp.int32),
        pltpu.VMEM((CHUNK,), jnp.float32),
        pltpu.SemaphoreType.DMA,
        pltpu.VMEM_SHARED((NPAD,), jnp.float32),
    ],
)
def _sc_degree(row_hbm, ones_hbm, zeros1_hbm, deg_out, idx_v, ones_v, sem,
               acc):
    _sc_deg_body(row_hbm, ones_hbm, zeros1_hbm, deg_out, idx_v, ones_v, sem,
                 acc)


def _sc_spmm_body(p_hbm, src_hbm, dst_hbm, zeros2_hbm, q_out,
                  idxs_v, idxd_v, rows_v, sem, acc):
    cid = lax.axis_index("c")
    sid = lax.axis_index("s")
    wid = sid * NC + cid
    pltpu.sync_copy(zeros2_hbm.at[pl.ds(sid * RPT, RPT)],
                    acc.at[pl.ds(sid * RPT, RPT)])
    plsc.subcore_barrier()

    ebase = wid * EPW

    def chunk(i, carry):
        base = ebase + i * CHUNK
        pltpu.sync_copy(src_hbm.at[pl.ds(base, CHUNK)], idxs_v)
        pltpu.sync_copy(dst_hbm.at[pl.ds(base, CHUNK)], idxd_v)
        pltpu.async_copy(p_hbm.at[idxs_v], rows_v, sem).wait()
        pltpu.sync_copy(rows_v, acc.at[idxd_v], add=True)
        return carry

    lax.fori_loop(0, NCHUNK, chunk, 0)
    plsc.subcore_barrier()
    pltpu.sync_copy(acc.at[pl.ds(sid * RPT, RPT)],
                    q_out.at[cid, pl.ds(sid * RPT, RPT)])


@functools.partial(
    pl.kernel,
    mesh=_SC_MESH,
    compiler_params=pltpu.CompilerParams(use_tc_tiling_on_sc=False),
    out_type=jax.ShapeDtypeStruct((NC, NPAD, D), jnp.float32),
    scratch_types=[
        pltpu.VMEM((CHUNK,), jnp.int32),
        pltpu.VMEM((CHUNK,), jnp.int32),
        pltpu.VMEM((CHUNK, D), jnp.float32),
        pltpu.SemaphoreType.DMA,
        pltpu.VMEM_SHARED((NPAD, D), jnp.float32),
    ],
)
def _sc_spmm(p_hbm, src_hbm, dst_hbm, zeros2_hbm, q_out,
             idxs_v, idxd_v, rows_v, sem, acc):
    _sc_spmm_body(p_hbm, src_hbm, dst_hbm, zeros2_hbm, q_out,
                  idxs_v, idxd_v, rows_v, sem, acc)


def _mlp_body(x_ref, w1t_ref, b1_ref, w2t_ref, b2_ref, o_ref):
    h = jnp.dot(x_ref[...], w1t_ref[...], preferred_element_type=jnp.float32)
    h = jnp.maximum(h + b1_ref[...], 0.0)
    o_ref[...] = (
        jnp.dot(h, w2t_ref[...], preferred_element_type=jnp.float32)
        + b2_ref[...]
    )


_mlp = pl.pallas_call(
    _mlp_body,
    grid=(GRID,),
    in_specs=[
        pl.BlockSpec((ROW_BLK, DF), lambda i: (i, 0)),
        pl.BlockSpec((DF, DF), lambda i: (0, 0)),
        pl.BlockSpec((1, DF), lambda i: (0, 0)),
        pl.BlockSpec((DF, D), lambda i: (0, 0)),
        pl.BlockSpec((1, D), lambda i: (0, 0)),
    ],
    out_specs=pl.BlockSpec((ROW_BLK, D), lambda i: (i, 0)),
    out_shape=jax.ShapeDtypeStruct((NPAD, D), jnp.float32),
)


def _prep_body(degp_ref, h_ref, d0_ref, dinvb_ref, p_ref, acc_ref):
    deg = degp_ref[0, :] + degp_ref[1, :]
    dinv = jnp.where(deg > 0, lax.rsqrt(deg), 0.0)
    dinvb = jnp.broadcast_to(dinv[:, None], (ROW_BLK, D))
    h = h_ref[...]
    dinvb_ref[...] = dinvb
    p_ref[...] = dinvb * h
    acc_ref[...] = d0_ref[0, 0] * h


_prep = pl.pallas_call(
    _prep_body,
    grid=(GRID,),
    in_specs=[
        pl.BlockSpec((2, ROW_BLK), lambda i: (0, i)),
        pl.BlockSpec((ROW_BLK, D), lambda i: (i, 0)),
        pl.BlockSpec((1, 1), lambda i: (0, 0)),
    ],
    out_specs=[
        pl.BlockSpec((ROW_BLK, D), lambda i: (i, 0)),
        pl.BlockSpec((ROW_BLK, D), lambda i: (i, 0)),
        pl.BlockSpec((ROW_BLK, D), lambda i: (i, 0)),
    ],
    out_shape=[
        jax.ShapeDtypeStruct((NPAD, D), jnp.float32),
        jax.ShapeDtypeStruct((NPAD, D), jnp.float32),
        jax.ShapeDtypeStruct((NPAD, D), jnp.float32),
    ],
)


def _combine_body(q_ref, dinvb_ref, accin_ref, dj_ref, accout_ref, pout_ref):
    dinvb = dinvb_ref[...]
    y = dinvb * (q_ref[0] + q_ref[1])
    accout_ref[...] = accin_ref[...] + dj_ref[0, 0] * y
    pout_ref[...] = dinvb * y


_combine = pl.pallas_call(
    _combine_body,
    grid=(GRID,),
    in_specs=[
        pl.BlockSpec((2, ROW_BLK, D), lambda i: (0, i, 0)),
        pl.BlockSpec((ROW_BLK, D), lambda i: (i, 0)),
        pl.BlockSpec((ROW_BLK, D), lambda i: (i, 0)),
        pl.BlockSpec((1, 1), lambda i: (0, 0)),
    ],
    out_specs=[
        pl.BlockSpec((ROW_BLK, D), lambda i: (i, 0)),
        pl.BlockSpec((ROW_BLK, D), lambda i: (i, 0)),
    ],
    out_shape=[
        jax.ShapeDtypeStruct((NPAD, D), jnp.float32),
        jax.ShapeDtypeStruct((NPAD, D), jnp.float32),
    ],
)


def _final_body(q_ref, dinvb_ref, accin_ref, dj_ref, out_ref):
    y = dinvb_ref[...] * (q_ref[0] + q_ref[1])
    acc = accin_ref[...] + dj_ref[0, 0] * y
    m = jnp.max(acc, axis=1, keepdims=True)
    lse = m + jnp.log(jnp.sum(jnp.exp(acc - m), axis=1, keepdims=True))
    out_ref[...] = acc - lse


_final = pl.pallas_call(
    _final_body,
    grid=(GRID,),
    in_specs=[
        pl.BlockSpec((2, ROW_BLK, D), lambda i: (0, i, 0)),
        pl.BlockSpec((ROW_BLK, D), lambda i: (i, 0)),
        pl.BlockSpec((ROW_BLK, D), lambda i: (i, 0)),
        pl.BlockSpec((1, 1), lambda i: (0, 0)),
    ],
    out_specs=pl.BlockSpec((ROW_BLK, D), lambda i: (i, 0)),
    out_shape=jax.ShapeDtypeStruct((NPAD, D), jnp.float32),
)


def kernel(x, edge_index, W1, b1, W2, b2, temp):
    row = edge_index[0].astype(jnp.int32)
    col = edge_index[1].astype(jnp.int32)
    pad = jnp.full((EPAD - E,), NPAD - 1, jnp.int32)
    src = jnp.concatenate([row, pad])
    dst = jnp.concatenate([col, pad])

    xpad = jnp.pad(x, ((0, NPAD - N), (0, 0)))
    ones_c = jnp.ones((CHUNK,), jnp.float32)
    zeros1 = jnp.zeros((NPAD,), jnp.float32)
    zeros2 = jnp.zeros((NPAD, D), jnp.float32)

    d = jnp.asarray(_BERN_M, jnp.float32) @ jax.nn.relu(temp)

    h = _mlp(xpad, W1.T, b1[None, :], W2.T, b2[None, :])
    degp = _sc_degree(src, ones_c, zeros1)
    dinvb, p, acc = _prep(degp, h, d[0].reshape(1, 1))
    for j in range(1, K + 1):
        q = _sc_spmm(p, src, dst, zeros2)
        if j < K:
            acc, p = _combine(q, dinvb, acc, d[j].reshape(1, 1))
        else:
            out = _final(q, dinvb, acc, d[j].reshape(1, 1))
    return out[:N]


# speedup vs baseline: 37.4844x; 1.4375x over previous
import functools
import math

import jax
import jax.numpy as jnp
import numpy as np
from jax import lax
from jax.experimental import pallas as pl
from jax.experimental.pallas import tpu as pltpu
from jax.experimental.pallas import tpu_sc as plsc

K = 10
N = 10000
D = 64
DF = 128
E = 320000

NC, NS = 2, 16
NW = NC * NS
RPT = 640
NPAD = RPT * NS
EPW = 10240
EPAD = EPW * NW
CHUNK = 128
NCHUNK = EPW // CHUNK

ROW_BLK = 1280
GRID = NPAD // ROW_BLK


def _bern_matrix():
    M = np.zeros((K + 1, K + 1), dtype=np.float64)
    for i in range(K + 1):
        for j in range(K + 1):
            g = 0
            for m in range(0, i + 1):
                if 0 <= j - m <= K - i:
                    g += (-1) ** m * math.comb(i, m) * math.comb(K - i, j - m)
            M[j, i] = (math.comb(K, i) / (2 ** K)) * g
    return M


_BERN_M = _bern_matrix()

_SC_MESH = plsc.VectorSubcoreMesh(core_axis_name="c", subcore_axis_name="s")


NBUF = 4
DEGK = 8


def _sc_deg_body(row_hbm, ones_hbm, zeros1_hbm, deg_out, idx_all, ones_v, sem,
                 acc):
    cid = lax.axis_index("c")
    sid = lax.axis_index("s")
    wid = sid * NC + cid
    pltpu.sync_copy(zeros1_hbm.at[pl.ds(sid * RPT, RPT)],
                    acc.at[pl.ds(sid * RPT, RPT)])
    pltpu.sync_copy(ones_hbm, ones_v)
    pltpu.sync_copy(row_hbm.at[wid], idx_all)
    plsc.subcore_barrier()

    def outer(t, carry):
        for b in range(DEGK):
            pltpu.async_copy(ones_v, acc.at[idx_all.at[t * DEGK + b]], sem,
                             add=True)
        for b in range(DEGK):
            pltpu.make_async_copy(ones_v, acc.at[idx_all.at[t * DEGK + b]],
                                  sem).wait()
        return carry

    lax.fori_loop(0, NCHUNK // DEGK, outer, 0)
    plsc.subcore_barrier()
    pltpu.sync_copy(acc.at[pl.ds(sid * RPT, RPT)],
                    deg_out.at[cid, pl.ds(sid * RPT, RPT)])


@functools.partial(
    pl.kernel,
    mesh=_SC_MESH,
    compiler_params=pltpu.CompilerParams(use_tc_tiling_on_sc=False),
    out_type=jax.ShapeDtypeStruct((NC, NPAD), jnp.float32),
    scratch_types=[
        pltpu.VMEM((NCHUNK, CHUNK), jnp.int32),
        pltpu.VMEM((CHUNK,), jnp.float32),
        pltpu.SemaphoreType.DMA,
        pltpu.VMEM_SHARED((NPAD,), jnp.float32),
    ],
)
def _sc_degree(row_hbm, ones_hbm, zeros1_hbm, deg_out, idx_all, ones_v, sem,
               acc):
    _sc_deg_body(row_hbm, ones_hbm, zeros1_hbm, deg_out, idx_all, ones_v, sem,
                 acc)


def _sc_spmm_body(p_hbm, src_hbm, dst_hbm, zeros2_hbm, q_out,
                  idxs_all, idxd_all, rows, gsems, acc):
    cid = lax.axis_index("c")
    sid = lax.axis_index("s")
    wid = sid * NC + cid
    pltpu.sync_copy(zeros2_hbm.at[pl.ds(sid * RPT, RPT)],
                    acc.at[pl.ds(sid * RPT, RPT)])
    pltpu.sync_copy(src_hbm.at[wid], idxs_all)
    pltpu.sync_copy(dst_hbm.at[wid], idxd_all)
    plsc.subcore_barrier()

    for b in range(NBUF):
        pltpu.async_copy(p_hbm.at[idxs_all.at[b]], rows[b], gsems[b])

    def outer(t, carry):
        for b in range(NBUF):
            i = t * NBUF + b
            pltpu.make_async_copy(p_hbm.at[idxs_all.at[i]], rows[b],
                                  gsems[b]).wait()
            pltpu.sync_copy(rows[b], acc.at[idxd_all.at[i]], add=True)

            @pl.when(i + NBUF < NCHUNK)
            def _prefetch():
                pltpu.async_copy(p_hbm.at[idxs_all.at[i + NBUF]], rows[b],
                                 gsems[b])

        return carry

    lax.fori_loop(0, NCHUNK // NBUF, outer, 0)
    plsc.subcore_barrier()
    pltpu.sync_copy(acc.at[pl.ds(sid * RPT, RPT)],
                    q_out.at[cid, pl.ds(sid * RPT, RPT)])


@functools.partial(
    pl.kernel,
    mesh=_SC_MESH,
    compiler_params=pltpu.CompilerParams(use_tc_tiling_on_sc=False),
    out_type=jax.ShapeDtypeStruct((NC, NPAD, D), jnp.float32),
    scratch_types=[
        pltpu.VMEM((NCHUNK, CHUNK), jnp.int32),
        pltpu.VMEM((NCHUNK, CHUNK), jnp.int32),
        [pltpu.VMEM((CHUNK, D), jnp.float32) for _ in range(NBUF)],
        [pltpu.SemaphoreType.DMA for _ in range(NBUF)],
        pltpu.VMEM_SHARED((NPAD, D), jnp.float32),
    ],
)
def _sc_spmm(p_hbm, src_hbm, dst_hbm, zeros2_hbm, q_out,
             idxs_all, idxd_all, rows, gsems, acc):
    _sc_spmm_body(p_hbm, src_hbm, dst_hbm, zeros2_hbm, q_out,
                  idxs_all, idxd_all, rows, gsems, acc)


def _mlp_body(x_ref, w1t_ref, b1_ref, w2t_ref, b2_ref, o_ref):
    h = jnp.dot(x_ref[...], w1t_ref[...], preferred_element_type=jnp.float32)
    h = jnp.maximum(h + b1_ref[...], 0.0)
    o_ref[...] = (
        jnp.dot(h, w2t_ref[...], preferred_element_type=jnp.float32)
        + b2_ref[...]
    )


_mlp = pl.pallas_call(
    _mlp_body,
    grid=(GRID,),
    in_specs=[
        pl.BlockSpec((ROW_BLK, DF), lambda i: (i, 0)),
        pl.BlockSpec((DF, DF), lambda i: (0, 0)),
        pl.BlockSpec((1, DF), lambda i: (0, 0)),
        pl.BlockSpec((DF, D), lambda i: (0, 0)),
        pl.BlockSpec((1, D), lambda i: (0, 0)),
    ],
    out_specs=pl.BlockSpec((ROW_BLK, D), lambda i: (i, 0)),
    out_shape=jax.ShapeDtypeStruct((NPAD, D), jnp.float32),
)


def _prep_body(degp_ref, h_ref, d0_ref, dinvb_ref, p_ref, acc_ref):
    deg = degp_ref[0, :] + degp_ref[1, :]
    dinv = jnp.where(deg > 0, lax.rsqrt(deg), 0.0)
    dinvb = jnp.broadcast_to(dinv[:, None], (ROW_BLK, D))
    h = h_ref[...]
    dinvb_ref[...] = dinvb
    p_ref[...] = dinvb * h
    acc_ref[...] = d0_ref[0, 0] * h


_prep = pl.pallas_call(
    _prep_body,
    grid=(GRID,),
    in_specs=[
        pl.BlockSpec((2, ROW_BLK), lambda i: (0, i)),
        pl.BlockSpec((ROW_BLK, D), lambda i: (i, 0)),
        pl.BlockSpec((1, 1), lambda i: (0, 0)),
    ],
    out_specs=[
        pl.BlockSpec((ROW_BLK, D), lambda i: (i, 0)),
        pl.BlockSpec((ROW_BLK, D), lambda i: (i, 0)),
        pl.BlockSpec((ROW_BLK, D), lambda i: (i, 0)),
    ],
    out_shape=[
        jax.ShapeDtypeStruct((NPAD, D), jnp.float32),
        jax.ShapeDtypeStruct((NPAD, D), jnp.float32),
        jax.ShapeDtypeStruct((NPAD, D), jnp.float32),
    ],
)


def _combine_body(q_ref, dinvb_ref, accin_ref, dj_ref, accout_ref, pout_ref):
    dinvb = dinvb_ref[...]
    y = dinvb * (q_ref[0] + q_ref[1])
    accout_ref[...] = accin_ref[...] + dj_ref[0, 0] * y
    pout_ref[...] = dinvb * y


_combine = pl.pallas_call(
    _combine_body,
    grid=(GRID,),
    in_specs=[
        pl.BlockSpec((2, ROW_BLK, D), lambda i: (0, i, 0)),
        pl.BlockSpec((ROW_BLK, D), lambda i: (i, 0)),
        pl.BlockSpec((ROW_BLK, D), lambda i: (i, 0)),
        pl.BlockSpec((1, 1), lambda i: (0, 0)),
    ],
    out_specs=[
        pl.BlockSpec((ROW_BLK, D), lambda i: (i, 0)),
        pl.BlockSpec((ROW_BLK, D), lambda i: (i, 0)),
    ],
    out_shape=[
        jax.ShapeDtypeStruct((NPAD, D), jnp.float32),
        jax.ShapeDtypeStruct((NPAD, D), jnp.float32),
    ],
)


def _final_body(q_ref, dinvb_ref, accin_ref, dj_ref, out_ref):
    y = dinvb_ref[...] * (q_ref[0] + q_ref[1])
    acc = accin_ref[...] + dj_ref[0, 0] * y
    m = jnp.max(acc, axis=1, keepdims=True)
    lse = m + jnp.log(jnp.sum(jnp.exp(acc - m), axis=1, keepdims=True))
    out_ref[...] = acc - lse


_final = pl.pallas_call(
    _final_body,
    grid=(GRID,),
    in_specs=[
        pl.BlockSpec((2, ROW_BLK, D), lambda i: (0, i, 0)),
        pl.BlockSpec((ROW_BLK, D), lambda i: (i, 0)),
        pl.BlockSpec((ROW_BLK, D), lambda i: (i, 0)),
        pl.BlockSpec((1, 1), lambda i: (0, 0)),
    ],
    out_specs=pl.BlockSpec((ROW_BLK, D), lambda i: (i, 0)),
    out_shape=jax.ShapeDtypeStruct((NPAD, D), jnp.float32),
)


def kernel(x, edge_index, W1, b1, W2, b2, temp):
    row = edge_index[0].astype(jnp.int32)
    col = edge_index[1].astype(jnp.int32)
    pad = jnp.full((EPAD - E,), NPAD - 1, jnp.int32)
    src = jnp.concatenate([row, pad]).reshape(NW, NCHUNK, CHUNK)
    dst = jnp.concatenate([col, pad]).reshape(NW, NCHUNK, CHUNK)

    xpad = jnp.pad(x, ((0, NPAD - N), (0, 0)))
    ones_c = jnp.ones((CHUNK,), jnp.float32)
    zeros1 = jnp.zeros((NPAD,), jnp.float32)
    zeros2 = jnp.zeros((NPAD, D), jnp.float32)

    d = jnp.asarray(_BERN_M, jnp.float32) @ jax.nn.relu(temp)

    h = _mlp(xpad, W1.T, b1[None, :], W2.T, b2[None, :])
    degp = _sc_degree(src, ones_c, zeros1)
    dinvb, p, acc = _prep(degp, h, d[0].reshape(1, 1))
    for j in range(1, K + 1):
        q = _sc_spmm(p, src, dst, zeros2)
        if j < K:
            acc, p = _combine(q, dinvb, acc, d[j].reshape(1, 1))
        else:
            out = _final(q, dinvb, acc, d[j].reshape(1, 1))
    return out[:N]


# speedup vs baseline: 37.6363x; 1.0041x over previous
import functools
import math

import jax
import jax.numpy as jnp
import numpy as np
from jax import lax
from jax.experimental import pallas as pl
from jax.experimental.pallas import tpu as pltpu
from jax.experimental.pallas import tpu_sc as plsc

K = 10
N = 10000
D = 64
DF = 128
E = 320000

NC, NS = 2, 16
NW = NC * NS
RPT = 640
NPAD = RPT * NS
EPW = 10240
EPAD = EPW * NW
CHUNK = 128
NCHUNK = EPW // CHUNK

ROW_BLK = 1280
GRID = NPAD // ROW_BLK


def _bern_matrix():
    M = np.zeros((K + 1, K + 1), dtype=np.float64)
    for i in range(K + 1):
        for j in range(K + 1):
            g = 0
            for m in range(0, i + 1):
                if 0 <= j - m <= K - i:
                    g += (-1) ** m * math.comb(i, m) * math.comb(K - i, j - m)
            M[j, i] = (math.comb(K, i) / (2 ** K)) * g
    return M


_BERN_M = _bern_matrix()

_SC_MESH = plsc.VectorSubcoreMesh(core_axis_name="c", subcore_axis_name="s")


NBUF = 4
DEGK = 8


def _sc_deg_body(row_hbm, ones_hbm, zeros1_hbm, deg_out, idx_all, ones_v, sem,
                 acc):
    cid = lax.axis_index("c")
    sid = lax.axis_index("s")
    wid = sid * NC + cid
    pltpu.sync_copy(zeros1_hbm.at[pl.ds(sid * RPT, RPT)],
                    acc.at[pl.ds(sid * RPT, RPT)])
    pltpu.sync_copy(ones_hbm, ones_v)
    pltpu.sync_copy(row_hbm.at[wid], idx_all)
    plsc.subcore_barrier()

    def outer(t, carry):
        for b in range(DEGK):
            pltpu.async_copy(ones_v, acc.at[idx_all.at[t * DEGK + b]], sem,
                             add=True)
        for b in range(DEGK):
            pltpu.make_async_copy(ones_v, acc.at[idx_all.at[t * DEGK + b]],
                                  sem).wait()
        return carry

    lax.fori_loop(0, NCHUNK // DEGK, outer, 0)
    plsc.subcore_barrier()
    pltpu.sync_copy(acc.at[pl.ds(sid * RPT, RPT)],
                    deg_out.at[cid, pl.ds(sid * RPT, RPT)])


@functools.partial(
    pl.kernel,
    mesh=_SC_MESH,
    compiler_params=pltpu.CompilerParams(use_tc_tiling_on_sc=False),
    out_type=jax.ShapeDtypeStruct((NC, NPAD), jnp.float32),
    scratch_types=[
        pltpu.VMEM((NCHUNK, CHUNK), jnp.int32),
        pltpu.VMEM((CHUNK,), jnp.float32),
        pltpu.SemaphoreType.DMA,
        pltpu.VMEM_SHARED((NPAD,), jnp.float32),
    ],
)
def _sc_degree(row_hbm, ones_hbm, zeros1_hbm, deg_out, idx_all, ones_v, sem,
               acc):
    _sc_deg_body(row_hbm, ones_hbm, zeros1_hbm, deg_out, idx_all, ones_v, sem,
                 acc)


PD = 4
NBUF2 = 8


def _sc_spmm_body(p_hbm, src_hbm, dst_hbm, zeros2_hbm, q_out,
                  idxs_all, idxd_all, rows, gsems, ssems, acc):
    cid = lax.axis_index("c")
    sid = lax.axis_index("s")
    wid = sid * NC + cid
    pltpu.sync_copy(zeros2_hbm.at[pl.ds(sid * RPT, RPT)],
                    acc.at[pl.ds(sid * RPT, RPT)])
    pltpu.sync_copy(src_hbm.at[wid], idxs_all)
    pltpu.sync_copy(dst_hbm.at[wid], idxd_all)
    plsc.subcore_barrier()

    def g_cp(i, b):
        return pltpu.make_async_copy(p_hbm.at[idxs_all.at[i]], rows[b],
                                     gsems[b])

    def s_cp(i, b):
        return pltpu.make_async_copy(rows[b], acc.at[idxd_all.at[i]],
                                     ssems[b])

    for i in range(PD):
        g_cp(i, i % NBUF2).start()

    def outer(t, carry):
        for b0 in range(NBUF2):
            j = t * NBUF2 + b0
            bp = (b0 + PD) % NBUF2

            @pl.when(j + PD < NCHUNK)
            def _prefetch():
                @pl.when(j + PD >= NBUF2)
                def _free():
                    s_cp(j + PD - NBUF2, bp).wait()

                pltpu.async_copy(p_hbm.at[idxs_all.at[j + PD]], rows[bp],
                                 gsems[bp])

            g_cp(j, b0).wait()
            pltpu.async_copy(rows[b0], acc.at[idxd_all.at[j]], ssems[b0],
                             add=True)
        return carry

    lax.fori_loop(0, NCHUNK // NBUF2, outer, 0)
    for i in range(NCHUNK - NBUF2, NCHUNK):
        s_cp(i, i % NBUF2).wait()
    plsc.subcore_barrier()
    pltpu.sync_copy(acc.at[pl.ds(sid * RPT, RPT)],
                    q_out.at[cid, pl.ds(sid * RPT, RPT)])


@functools.partial(
    pl.kernel,
    mesh=_SC_MESH,
    compiler_params=pltpu.CompilerParams(use_tc_tiling_on_sc=False),
    out_type=jax.ShapeDtypeStruct((NC, NPAD, D), jnp.float32),
    scratch_types=[
        pltpu.VMEM((NCHUNK, CHUNK), jnp.int32),
        pltpu.VMEM((NCHUNK, CHUNK), jnp.int32),
        [pltpu.VMEM((CHUNK, D), jnp.float32) for _ in range(NBUF2)],
        [pltpu.SemaphoreType.DMA for _ in range(NBUF2)],
        [pltpu.SemaphoreType.DMA for _ in range(NBUF2)],
        pltpu.VMEM_SHARED((NPAD, D), jnp.float32),
    ],
)
def _sc_spmm(p_hbm, src_hbm, dst_hbm, zeros2_hbm, q_out,
             idxs_all, idxd_all, rows, gsems, ssems, acc):
    _sc_spmm_body(p_hbm, src_hbm, dst_hbm, zeros2_hbm, q_out,
                  idxs_all, idxd_all, rows, gsems, ssems, acc)


def _mlp_body(x_ref, w1t_ref, b1_ref, w2t_ref, b2_ref, o_ref):
    h = jnp.dot(x_ref[...], w1t_ref[...], preferred_element_type=jnp.float32)
    h = jnp.maximum(h + b1_ref[...], 0.0)
    o_ref[...] = (
        jnp.dot(h, w2t_ref[...], preferred_element_type=jnp.float32)
        + b2_ref[...]
    )


_mlp = pl.pallas_call(
    _mlp_body,
    grid=(GRID,),
    in_specs=[
        pl.BlockSpec((ROW_BLK, DF), lambda i: (i, 0)),
        pl.BlockSpec((DF, DF), lambda i: (0, 0)),
        pl.BlockSpec((1, DF), lambda i: (0, 0)),
        pl.BlockSpec((DF, D), lambda i: (0, 0)),
        pl.BlockSpec((1, D), lambda i: (0, 0)),
    ],
    out_specs=pl.BlockSpec((ROW_BLK, D), lambda i: (i, 0)),
    out_shape=jax.ShapeDtypeStruct((NPAD, D), jnp.float32),
)


def _prep_body(degp_ref, h_ref, d0_ref, dinvb_ref, p_ref, acc_ref):
    deg = degp_ref[0, :] + degp_ref[1, :]
    dinv = jnp.where(deg > 0, lax.rsqrt(deg), 0.0)
    dinvb = jnp.broadcast_to(dinv[:, None], (ROW_BLK, D))
    h = h_ref[...]
    dinvb_ref[...] = dinvb
    p_ref[...] = dinvb * h
    acc_ref[...] = d0_ref[0, 0] * h


_prep = pl.pallas_call(
    _prep_body,
    grid=(GRID,),
    in_specs=[
        pl.BlockSpec((2, ROW_BLK), lambda i: (0, i)),
        pl.BlockSpec((ROW_BLK, D), lambda i: (i, 0)),
        pl.BlockSpec((1, 1), lambda i: (0, 0)),
    ],
    out_specs=[
        pl.BlockSpec((ROW_BLK, D), lambda i: (i, 0)),
        pl.BlockSpec((ROW_BLK, D), lambda i: (i, 0)),
        pl.BlockSpec((ROW_BLK, D), lambda i: (i, 0)),
    ],
    out_shape=[
        jax.ShapeDtypeStruct((NPAD, D), jnp.float32),
        jax.ShapeDtypeStruct((NPAD, D), jnp.float32),
        jax.ShapeDtypeStruct((NPAD, D), jnp.float32),
    ],
)


def _combine_body(q_ref, dinvb_ref, accin_ref, dj_ref, accout_ref, pout_ref):
    dinvb = dinvb_ref[...]
    y = dinvb * (q_ref[0] + q_ref[1])
    accout_ref[...] = accin_ref[...] + dj_ref[0, 0] * y
    pout_ref[...] = dinvb * y


_combine = pl.pallas_call(
    _combine_body,
    grid=(GRID,),
    in_specs=[
        pl.BlockSpec((2, ROW_BLK, D), lambda i: (0, i, 0)),
        pl.BlockSpec((ROW_BLK, D), lambda i: (i, 0)),
        pl.BlockSpec((ROW_BLK, D), lambda i: (i, 0)),
        pl.BlockSpec((1, 1), lambda i: (0, 0)),
    ],
    out_specs=[
        pl.BlockSpec((ROW_BLK, D), lambda i: (i, 0)),
        pl.BlockSpec((ROW_BLK, D), lambda i: (i, 0)),
    ],
    out_shape=[
        jax.ShapeDtypeStruct((NPAD, D), jnp.float32),
        jax.ShapeDtypeStruct((NPAD, D), jnp.float32),
    ],
)


def _final_body(q_ref, dinvb_ref, accin_ref, dj_ref, out_ref):
    y = dinvb_ref[...] * (q_ref[0] + q_ref[1])
    acc = accin_ref[...] + dj_ref[0, 0] * y
    m = jnp.max(acc, axis=1, keepdims=True)
    lse = m + jnp.log(jnp.sum(jnp.exp(acc - m), axis=1, keepdims=True))
    out_ref[...] = acc - lse


_final = pl.pallas_call(
    _final_body,
    grid=(GRID,),
    in_specs=[
        pl.BlockSpec((2, ROW_BLK, D), lambda i: (0, i, 0)),
        pl.BlockSpec((ROW_BLK, D), lambda i: (i, 0)),
        pl.BlockSpec((ROW_BLK, D), lambda i: (i, 0)),
        pl.BlockSpec((1, 1), lambda i: (0, 0)),
    ],
    out_specs=pl.BlockSpec((ROW_BLK, D), lambda i: (i, 0)),
    out_shape=jax.ShapeDtypeStruct((NPAD, D), jnp.float32),
)


def kernel(x, edge_index, W1, b1, W2, b2, temp):
    row = edge_index[0].astype(jnp.int32)
    col = edge_index[1].astype(jnp.int32)
    pad = jnp.full((EPAD - E,), NPAD - 1, jnp.int32)
    src = jnp.concatenate([row, pad]).reshape(NW, NCHUNK, CHUNK)
    dst = jnp.concatenate([col, pad]).reshape(NW, NCHUNK, CHUNK)

    xpad = jnp.pad(x, ((0, NPAD - N), (0, 0)))
    ones_c = jnp.ones((CHUNK,), jnp.float32)
    zeros1 = jnp.zeros((NPAD,), jnp.float32)
    zeros2 = jnp.zeros((NPAD, D), jnp.float32)

    d = jnp.asarray(_BERN_M, jnp.float32) @ jax.nn.relu(temp)

    h = _mlp(xpad, W1.T, b1[None, :], W2.T, b2[None, :])
    degp = _sc_degree(src, ones_c, zeros1)
    dinvb, p, acc = _prep(degp, h, d[0].reshape(1, 1))
    for j in range(1, K + 1):
        q = _sc_spmm(p, src, dst, zeros2)
        if j < K:
            acc, p = _combine(q, dinvb, acc, d[j].reshape(1, 1))
        else:
            out = _final(q, dinvb, acc, d[j].reshape(1, 1))
    return out[:N]


# speedup vs baseline: 87.1791x; 2.3164x over previous
import functools
import math

import jax
import jax.numpy as jnp
import numpy as np
from jax import lax
from jax.experimental import pallas as pl
from jax.experimental.pallas import tpu as pltpu
from jax.experimental.pallas import tpu_sc as plsc

K = 10
N = 10000
D = 64
DF = 128
E = 320000

NC, NS = 2, 16
NW = NC * NS
RPT = 640
NPAD = RPT * NS
EPW = 10240
EPAD = EPW * NW
CHUNK = 128
NCHUNK = EPW // CHUNK

ROW_BLK = 1280
GRID = NPAD // ROW_BLK


def _bern_matrix():
    M = np.zeros((K + 1, K + 1), dtype=np.float64)
    for i in range(K + 1):
        for j in range(K + 1):
            g = 0
            for m in range(0, i + 1):
                if 0 <= j - m <= K - i:
                    g += (-1) ** m * math.comb(i, m) * math.comb(K - i, j - m)
            M[j, i] = (math.comb(K, i) / (2 ** K)) * g
    return M


_BERN_M = _bern_matrix()

_SC_MESH = plsc.VectorSubcoreMesh(core_axis_name="c", subcore_axis_name="s")


NBUF = 4
DEGK = 8


def _sc_deg_body(row_hbm, ones_hbm, zeros1_hbm, deg_out, idx_all, ones_v, sem,
                 acc):
    cid = lax.axis_index("c")
    sid = lax.axis_index("s")
    wid = sid * NC + cid
    pltpu.sync_copy(zeros1_hbm.at[pl.ds(sid * RPT, RPT)],
                    acc.at[pl.ds(sid * RPT, RPT)])
    pltpu.sync_copy(ones_hbm, ones_v)
    pltpu.sync_copy(row_hbm.at[wid], idx_all)
    plsc.subcore_barrier()

    def outer(t, carry):
        for b in range(DEGK):
            pltpu.async_copy(ones_v, acc.at[idx_all.at[t * DEGK + b]], sem,
                             add=True)
        for b in range(DEGK):
            pltpu.make_async_copy(ones_v, acc.at[idx_all.at[t * DEGK + b]],
                                  sem).wait()
        return carry

    lax.fori_loop(0, NCHUNK // DEGK, outer, 0)
    plsc.subcore_barrier()
    pltpu.sync_copy(acc.at[pl.ds(sid * RPT, RPT)],
                    deg_out.at[cid, pl.ds(sid * RPT, RPT)])


@functools.partial(
    pl.kernel,
    mesh=_SC_MESH,
    compiler_params=pltpu.CompilerParams(use_tc_tiling_on_sc=False),
    out_type=jax.ShapeDtypeStruct((NC, NPAD), jnp.float32),
    scratch_types=[
        pltpu.VMEM((NCHUNK, CHUNK), jnp.int32),
        pltpu.VMEM((CHUNK,), jnp.float32),
        pltpu.SemaphoreType.DMA,
        pltpu.VMEM_SHARED((NPAD,), jnp.float32),
    ],
)
def _sc_degree(row_hbm, ones_hbm, zeros1_hbm, deg_out, idx_all, ones_v, sem,
               acc):
    _sc_deg_body(row_hbm, ones_hbm, zeros1_hbm, deg_out, idx_all, ones_v, sem,
                 acc)


PD = 2
NBUF2 = 4
HALF = NCHUNK // 2


def _sc_spmm_body(p_hbm, src_hbm, dst_hbm, zeros2_hbm, q_out,
                  idxs_all, idxd_all, rows, gsems, ssems, pstage, acc):
    cid = lax.axis_index("c")
    sid = lax.axis_index("s")
    wid = sid * NC + cid
    pltpu.sync_copy(zeros2_hbm.at[pl.ds(sid * RPT, RPT)],
                    acc.at[pl.ds(sid * RPT, RPT)])
    pltpu.sync_copy(p_hbm.at[pl.ds(sid * RPT, RPT)],
                    pstage.at[pl.ds(sid * RPT, RPT)])
    plsc.subcore_barrier()

    def g_cp(i, b):
        return pltpu.make_async_copy(pstage.at[idxs_all.at[i]], rows[b],
                                     gsems[b])

    def s_cp(i, b):
        return pltpu.make_async_copy(rows[b], acc.at[idxd_all.at[i]],
                                     ssems[b])

    for ph in range(2):
        pltpu.sync_copy(src_hbm.at[wid, pl.ds(ph * HALF, HALF)], idxs_all)
        pltpu.sync_copy(dst_hbm.at[wid, pl.ds(ph * HALF, HALF)], idxd_all)
        for i in range(PD):
            g_cp(i, i % NBUF2).start()

        def outer(t, carry):
            for b0 in range(NBUF2):
                j = t * NBUF2 + b0
                bp = (b0 + PD) % NBUF2

                @pl.when(j + PD < HALF)
                def _prefetch():
                    @pl.when(j + PD >= NBUF2)
                    def _free():
                        s_cp(j + PD - NBUF2, bp).wait()

                    pltpu.async_copy(pstage.at[idxs_all.at[j + PD]],
                                     rows[bp], gsems[bp])

                g_cp(j, b0).wait()
                pltpu.async_copy(rows[b0], acc.at[idxd_all.at[j]], ssems[b0],
                                 add=True)
            return carry

        lax.fori_loop(0, HALF // NBUF2, outer, 0)
        for i in range(HALF - NBUF2, HALF):
            s_cp(i, i % NBUF2).wait()

    plsc.subcore_barrier()
    pltpu.sync_copy(acc.at[pl.ds(sid * RPT, RPT)],
                    q_out.at[cid, pl.ds(sid * RPT, RPT)])


@functools.partial(
    pl.kernel,
    mesh=_SC_MESH,
    compiler_params=pltpu.CompilerParams(use_tc_tiling_on_sc=False),
    out_type=jax.ShapeDtypeStruct((NC, NPAD, D), jnp.float32),
    scratch_types=[
        pltpu.VMEM((HALF, CHUNK), jnp.int32),
        pltpu.VMEM((HALF, CHUNK), jnp.int32),
        [pltpu.VMEM((CHUNK, D), jnp.float32) for _ in range(NBUF2)],
        [pltpu.SemaphoreType.DMA for _ in range(NBUF2)],
        [pltpu.SemaphoreType.DMA for _ in range(NBUF2)],
        pltpu.VMEM_SHARED((NPAD, D), jnp.float32),
        pltpu.VMEM_SHARED((NPAD, D), jnp.float32),
    ],
)
def _sc_spmm(p_hbm, src_hbm, dst_hbm, zeros2_hbm, q_out,
             idxs_all, idxd_all, rows, gsems, ssems, pstage, acc):
    _sc_spmm_body(p_hbm, src_hbm, dst_hbm, zeros2_hbm, q_out,
                  idxs_all, idxd_all, rows, gsems, ssems, pstage, acc)


def _mlp_body(x_ref, w1t_ref, b1_ref, w2t_ref, b2_ref, o_ref):
    h = jnp.dot(x_ref[...], w1t_ref[...], preferred_element_type=jnp.float32)
    h = jnp.maximum(h + b1_ref[...], 0.0)
    o_ref[...] = (
        jnp.dot(h, w2t_ref[...], preferred_element_type=jnp.float32)
        + b2_ref[...]
    )


_mlp = pl.pallas_call(
    _mlp_body,
    grid=(GRID,),
    in_specs=[
        pl.BlockSpec((ROW_BLK, DF), lambda i: (i, 0)),
        pl.BlockSpec((DF, DF), lambda i: (0, 0)),
        pl.BlockSpec((1, DF), lambda i: (0, 0)),
        pl.BlockSpec((DF, D), lambda i: (0, 0)),
        pl.BlockSpec((1, D), lambda i: (0, 0)),
    ],
    out_specs=pl.BlockSpec((ROW_BLK, D), lambda i: (i, 0)),
    out_shape=jax.ShapeDtypeStruct((NPAD, D), jnp.float32),
)


def _prep_body(degp_ref, h_ref, d0_ref, dinvb_ref, p_ref, acc_ref):
    deg = degp_ref[0, :] + degp_ref[1, :]
    dinv = jnp.where(deg > 0, lax.rsqrt(deg), 0.0)
    dinvb = jnp.broadcast_to(dinv[:, None], (ROW_BLK, D))
    h = h_ref[...]
    dinvb_ref[...] = dinvb
    p_ref[...] = dinvb * h
    acc_ref[...] = d0_ref[0, 0] * h


_prep = pl.pallas_call(
    _prep_body,
    grid=(GRID,),
    in_specs=[
        pl.BlockSpec((2, ROW_BLK), lambda i: (0, i)),
        pl.BlockSpec((ROW_BLK, D), lambda i: (i, 0)),
        pl.BlockSpec((1, 1), lambda i: (0, 0)),
    ],
    out_specs=[
        pl.BlockSpec((ROW_BLK, D), lambda i: (i, 0)),
        pl.BlockSpec((ROW_BLK, D), lambda i: (i, 0)),
        pl.BlockSpec((ROW_BLK, D), lambda i: (i, 0)),
    ],
    out_shape=[
        jax.ShapeDtypeStruct((NPAD, D), jnp.float32),
        jax.ShapeDtypeStruct((NPAD, D), jnp.float32),
        jax.ShapeDtypeStruct((NPAD, D), jnp.float32),
    ],
)


def _combine_body(q_ref, dinvb_ref, accin_ref, dj_ref, accout_ref, pout_ref):
    dinvb = dinvb_ref[...]
    y = dinvb * (q_ref[0] + q_ref[1])
    accout_ref[...] = accin_ref[...] + dj_ref[0, 0] * y
    pout_ref[...] = dinvb * y


_combine = pl.pallas_call(
    _combine_body,
    grid=(GRID,),
    in_specs=[
        pl.BlockSpec((2, ROW_BLK, D), lambda i: (0, i, 0)),
        pl.BlockSpec((ROW_BLK, D), lambda i: (i, 0)),
        pl.BlockSpec((ROW_BLK, D), lambda i: (i, 0)),
        pl.BlockSpec((1, 1), lambda i: (0, 0)),
    ],
    out_specs=[
        pl.BlockSpec((ROW_BLK, D), lambda i: (i, 0)),
        pl.BlockSpec((ROW_BLK, D), lambda i: (i, 0)),
    ],
    out_shape=[
        jax.ShapeDtypeStruct((NPAD, D), jnp.float32),
        jax.ShapeDtypeStruct((NPAD, D), jnp.float32),
    ],
)


def _final_body(q_ref, dinvb_ref, accin_ref, dj_ref, out_ref):
    y = dinvb_ref[...] * (q_ref[0] + q_ref[1])
    acc = accin_ref[...] + dj_ref[0, 0] * y
    m = jnp.max(acc, axis=1, keepdims=True)
    lse = m + jnp.log(jnp.sum(jnp.exp(acc - m), axis=1, keepdims=True))
    out_ref[...] = acc - lse


_final = pl.pallas_call(
    _final_body,
    grid=(GRID,),
    in_specs=[
        pl.BlockSpec((2, ROW_BLK, D), lambda i: (0, i, 0)),
        pl.BlockSpec((ROW_BLK, D), lambda i: (i, 0)),
        pl.BlockSpec((ROW_BLK, D), lambda i: (i, 0)),
        pl.BlockSpec((1, 1), lambda i: (0, 0)),
    ],
    out_specs=pl.BlockSpec((ROW_BLK, D), lambda i: (i, 0)),
    out_shape=jax.ShapeDtypeStruct((NPAD, D), jnp.float32),
)


def kernel(x, edge_index, W1, b1, W2, b2, temp):
    row = edge_index[0].astype(jnp.int32)
    col = edge_index[1].astype(jnp.int32)
    pad = jnp.full((EPAD - E,), NPAD - 1, jnp.int32)
    src = jnp.concatenate([row, pad]).reshape(NW, NCHUNK, CHUNK)
    dst = jnp.concatenate([col, pad]).reshape(NW, NCHUNK, CHUNK)

    xpad = jnp.pad(x, ((0, NPAD - N), (0, 0)))
    ones_c = jnp.ones((CHUNK,), jnp.float32)
    zeros1 = jnp.zeros((NPAD,), jnp.float32)
    zeros2 = jnp.zeros((NPAD, D), jnp.float32)

    d = jnp.asarray(_BERN_M, jnp.float32) @ jax.nn.relu(temp)

    h = _mlp(xpad, W1.T, b1[None, :], W2.T, b2[None, :])
    degp = _sc_degree(src, ones_c, zeros1)
    dinvb, p, acc = _prep(degp, h, d[0].reshape(1, 1))
    for j in range(1, K + 1):
        q = _sc_spmm(p, src, dst, zeros2)
        if j < K:
            acc, p = _combine(q, dinvb, acc, d[j].reshape(1, 1))
        else:
            out = _final(q, dinvb, acc, d[j].reshape(1, 1))
    return out[:N]


# speedup vs baseline: 87.2185x; 1.0005x over previous
import functools
import math

import jax
import jax.numpy as jnp
import numpy as np
from jax import lax
from jax.experimental import pallas as pl
from jax.experimental.pallas import tpu as pltpu
from jax.experimental.pallas import tpu_sc as plsc

K = 10
N = 10000
D = 64
DF = 128
E = 320000

NC, NS = 2, 16
NW = NC * NS
RPT = 640
NPAD = RPT * NS
EPW = 10240
EPAD = EPW * NW
CHUNK = 128
NCHUNK = EPW // CHUNK

ROW_BLK = 1280
GRID = NPAD // ROW_BLK


def _bern_matrix():
    M = np.zeros((K + 1, K + 1), dtype=np.float64)
    for i in range(K + 1):
        for j in range(K + 1):
            g = 0
            for m in range(0, i + 1):
                if 0 <= j - m <= K - i:
                    g += (-1) ** m * math.comb(i, m) * math.comb(K - i, j - m)
            M[j, i] = (math.comb(K, i) / (2 ** K)) * g
    return M


_BERN_M = _bern_matrix()

_SC_MESH = plsc.VectorSubcoreMesh(core_axis_name="c", subcore_axis_name="s")


NBUF = 4
DEGK = 8


def _sc_deg_body(row_hbm, ones_hbm, zeros1_hbm, deg_out, idx_all, ones_v, sem,
                 acc):
    cid = lax.axis_index("c")
    sid = lax.axis_index("s")
    wid = sid * NC + cid
    pltpu.sync_copy(zeros1_hbm.at[pl.ds(sid * RPT, RPT)],
                    acc.at[pl.ds(sid * RPT, RPT)])
    pltpu.sync_copy(ones_hbm, ones_v)
    pltpu.sync_copy(row_hbm.at[wid], idx_all)
    plsc.subcore_barrier()

    def outer(t, carry):
        for b in range(DEGK):
            pltpu.async_copy(ones_v, acc.at[idx_all.at[t * DEGK + b]], sem,
                             add=True)
        for b in range(DEGK):
            pltpu.make_async_copy(ones_v, acc.at[idx_all.at[t * DEGK + b]],
                                  sem).wait()
        return carry

    lax.fori_loop(0, NCHUNK // DEGK, outer, 0)
    plsc.subcore_barrier()
    pltpu.sync_copy(acc.at[pl.ds(sid * RPT, RPT)],
                    deg_out.at[cid, pl.ds(sid * RPT, RPT)])


@functools.partial(
    pl.kernel,
    mesh=_SC_MESH,
    compiler_params=pltpu.CompilerParams(use_tc_tiling_on_sc=False),
    out_type=jax.ShapeDtypeStruct((NC, NPAD), jnp.float32),
    scratch_types=[
        pltpu.VMEM((NCHUNK, CHUNK), jnp.int32),
        pltpu.VMEM((CHUNK,), jnp.float32),
        pltpu.SemaphoreType.DMA,
        pltpu.VMEM_SHARED((NPAD,), jnp.float32),
    ],
)
def _sc_degree(row_hbm, ones_hbm, zeros1_hbm, deg_out, idx_all, ones_v, sem,
               acc):
    _sc_deg_body(row_hbm, ones_hbm, zeros1_hbm, deg_out, idx_all, ones_v, sem,
                 acc)


PD = 2
NBUF2 = 4
HALF = NCHUNK // 2
NH = 20


def _sc_spmm_body(p_hbm, src_hbm, dst_hbm, zeros2_hbm, q_out,
                  idxs_all, idxd_all, rows, gsems, ssems, pstage, acc):
    cid = lax.axis_index("c")
    sid = lax.axis_index("s")
    wid = sid * NC + cid
    pltpu.sync_copy(zeros2_hbm.at[pl.ds(sid * RPT, RPT)],
                    acc.at[pl.ds(sid * RPT, RPT)])
    pltpu.sync_copy(p_hbm.at[pl.ds(sid * RPT, RPT)],
                    pstage.at[pl.ds(sid * RPT, RPT)])
    plsc.subcore_barrier()

    def g_cp(i, b):
        return pltpu.make_async_copy(pstage.at[idxs_all.at[i]], rows[b],
                                     gsems[b])

    def s_cp(i, b):
        return pltpu.make_async_copy(rows[b], acc.at[idxd_all.at[i]],
                                     ssems[b])

    for ph in range(2):
        pltpu.sync_copy(src_hbm.at[wid, pl.ds(ph * HALF, HALF)], idxs_all)
        pltpu.sync_copy(dst_hbm.at[wid, pl.ds(ph * HALF, HALF)], idxd_all)

        def issue_gather(j, b):
            if ph == 0:
                j = jnp.int32(j)

                @pl.when(j < NH)
                def _hbm():
                    pltpu.async_copy(p_hbm.at[idxs_all.at[j]], rows[b],
                                     gsems[b])

                @pl.when(j >= NH)
                def _spm():
                    pltpu.async_copy(pstage.at[idxs_all.at[j]], rows[b],
                                     gsems[b])
            else:
                pltpu.async_copy(pstage.at[idxs_all.at[j]], rows[b],
                                 gsems[b])

        for i in range(PD):
            issue_gather(i, i % NBUF2)

        def outer(t, carry):
            for b0 in range(NBUF2):
                j = t * NBUF2 + b0
                bp = (b0 + PD) % NBUF2

                @pl.when(j + PD < HALF)
                def _prefetch():
                    @pl.when(j + PD >= NBUF2)
                    def _free():
                        s_cp(j + PD - NBUF2, bp).wait()

                    issue_gather(j + PD, bp)

                g_cp(j, b0).wait()
                pltpu.async_copy(rows[b0], acc.at[idxd_all.at[j]], ssems[b0],
                                 add=True)
            return carry

        lax.fori_loop(0, HALF // NBUF2, outer, 0)
        for i in range(HALF - NBUF2, HALF):
            s_cp(i, i % NBUF2).wait()

    plsc.subcore_barrier()
    pltpu.sync_copy(acc.at[pl.ds(sid * RPT, RPT)],
                    q_out.at[cid, pl.ds(sid * RPT, RPT)])


@functools.partial(
    pl.kernel,
    mesh=_SC_MESH,
    compiler_params=pltpu.CompilerParams(use_tc_tiling_on_sc=False),
    out_type=jax.ShapeDtypeStruct((NC, NPAD, D), jnp.float32),
    scratch_types=[
        pltpu.VMEM((HALF, CHUNK), jnp.int32),
        pltpu.VMEM((HALF, CHUNK), jnp.int32),
        [pltpu.VMEM((CHUNK, D), jnp.float32) for _ in range(NBUF2)],
        [pltpu.SemaphoreType.DMA for _ in range(NBUF2)],
        [pltpu.SemaphoreType.DMA for _ in range(NBUF2)],
        pltpu.VMEM_SHARED((NPAD, D), jnp.float32),
        pltpu.VMEM_SHARED((NPAD, D), jnp.float32),
    ],
)
def _sc_spmm(p_hbm, src_hbm, dst_hbm, zeros2_hbm, q_out,
             idxs_all, idxd_all, rows, gsems, ssems, pstage, acc):
    _sc_spmm_body(p_hbm, src_hbm, dst_hbm, zeros2_hbm, q_out,
                  idxs_all, idxd_all, rows, gsems, ssems, pstage, acc)


def _mlp_body(x_ref, w1t_ref, b1_ref, w2t_ref, b2_ref, o_ref):
    h = jnp.dot(x_ref[...], w1t_ref[...], preferred_element_type=jnp.float32)
    h = jnp.maximum(h + b1_ref[...], 0.0)
    o_ref[...] = (
        jnp.dot(h, w2t_ref[...], preferred_element_type=jnp.float32)
        + b2_ref[...]
    )


_mlp = pl.pallas_call(
    _mlp_body,
    grid=(GRID,),
    in_specs=[
        pl.BlockSpec((ROW_BLK, DF), lambda i: (i, 0)),
        pl.BlockSpec((DF, DF), lambda i: (0, 0)),
        pl.BlockSpec((1, DF), lambda i: (0, 0)),
        pl.BlockSpec((DF, D), lambda i: (0, 0)),
        pl.BlockSpec((1, D), lambda i: (0, 0)),
    ],
    out_specs=pl.BlockSpec((ROW_BLK, D), lambda i: (i, 0)),
    out_shape=jax.ShapeDtypeStruct((NPAD, D), jnp.float32),
)


def _prep_body(degp_ref, h_ref, d0_ref, dinvb_ref, p_ref, acc_ref):
    deg = degp_ref[0, :] + degp_ref[1, :]
    dinv = jnp.where(deg > 0, lax.rsqrt(deg), 0.0)
    dinvb = jnp.broadcast_to(dinv[:, None], (ROW_BLK, D))
    h = h_ref[...]
    dinvb_ref[...] = dinvb
    p_ref[...] = dinvb * h
    acc_ref[...] = d0_ref[0, 0] * h


_prep = pl.pallas_call(
    _prep_body,
    grid=(GRID,),
    in_specs=[
        pl.BlockSpec((2, ROW_BLK), lambda i: (0, i)),
        pl.BlockSpec((ROW_BLK, D), lambda i: (i, 0)),
        pl.BlockSpec((1, 1), lambda i: (0, 0)),
    ],
    out_specs=[
        pl.BlockSpec((ROW_BLK, D), lambda i: (i, 0)),
        pl.BlockSpec((ROW_BLK, D), lambda i: (i, 0)),
        pl.BlockSpec((ROW_BLK, D), lambda i: (i, 0)),
    ],
    out_shape=[
        jax.ShapeDtypeStruct((NPAD, D), jnp.float32),
        jax.ShapeDtypeStruct((NPAD, D), jnp.float32),
        jax.ShapeDtypeStruct((NPAD, D), jnp.float32),
    ],
)


def _combine_body(q_ref, dinvb_ref, accin_ref, dj_ref, accout_ref, pout_ref):
    dinvb = dinvb_ref[...]
    y = dinvb * (q_ref[0] + q_ref[1])
    accout_ref[...] = accin_ref[...] + dj_ref[0, 0] * y
    pout_ref[...] = dinvb * y


_combine = pl.pallas_call(
    _combine_body,
    grid=(GRID,),
    in_specs=[
        pl.BlockSpec((2, ROW_BLK, D), lambda i: (0, i, 0)),
        pl.BlockSpec((ROW_BLK, D), lambda i: (i, 0)),
        pl.BlockSpec((ROW_BLK, D), lambda i: (i, 0)),
        pl.BlockSpec((1, 1), lambda i: (0, 0)),
    ],
    out_specs=[
        pl.BlockSpec((ROW_BLK, D), lambda i: (i, 0)),
        pl.BlockSpec((ROW_BLK, D), lambda i: (i, 0)),
    ],
    out_shape=[
        jax.ShapeDtypeStruct((NPAD, D), jnp.float32),
        jax.ShapeDtypeStruct((NPAD, D), jnp.float32),
    ],
)


def _final_body(q_ref, dinvb_ref, accin_ref, dj_ref, out_ref):
    y = dinvb_ref[...] * (q_ref[0] + q_ref[1])
    acc = accin_ref[...] + dj_ref[0, 0] * y
    m = jnp.max(acc, axis=1, keepdims=True)
    lse = m + jnp.log(jnp.sum(jnp.exp(acc - m), axis=1, keepdims=True))
    out_ref[...] = acc - lse


_final = pl.pallas_call(
    _final_body,
    grid=(GRID,),
    in_specs=[
        pl.BlockSpec((2, ROW_BLK, D), lambda i: (0, i, 0)),
        pl.BlockSpec((ROW_BLK, D), lambda i: (i, 0)),
        pl.BlockSpec((ROW_BLK, D), lambda i: (i, 0)),
        pl.BlockSpec((1, 1), lambda i: (0, 0)),
    ],
    out_specs=pl.BlockSpec((ROW_BLK, D), lambda i: (i, 0)),
    out_shape=jax.ShapeDtypeStruct((NPAD, D), jnp.float32),
)


def kernel(x, edge_index, W1, b1, W2, b2, temp):
    row = edge_index[0].astype(jnp.int32)
    col = edge_index[1].astype(jnp.int32)
    pad = jnp.full((EPAD - E,), NPAD - 1, jnp.int32)
    src = jnp.concatenate([row, pad]).reshape(NW, NCHUNK, CHUNK)
    dst = jnp.concatenate([col, pad]).reshape(NW, NCHUNK, CHUNK)

    xpad = jnp.pad(x, ((0, NPAD - N), (0, 0)))
    ones_c = jnp.ones((CHUNK,), jnp.float32)
    zeros1 = jnp.zeros((NPAD,), jnp.float32)
    zeros2 = jnp.zeros((NPAD, D), jnp.float32)

    d = jnp.asarray(_BERN_M, jnp.float32) @ jax.nn.relu(temp)

    h = _mlp(xpad, W1.T, b1[None, :], W2.T, b2[None, :])
    degp = _sc_degree(src, ones_c, zeros1)
    dinvb, p, acc = _prep(degp, h, d[0].reshape(1, 1))
    for j in range(1, K + 1):
        q = _sc_spmm(p, src, dst, zeros2)
        if j < K:
            acc, p = _combine(q, dinvb, acc, d[j].reshape(1, 1))
        else:
            out = _final(q, dinvb, acc, d[j].reshape(1, 1))
    return out[:N]


# speedup vs baseline: 91.2101x; 1.0458x over previous
import functools
import math

import jax
import jax.numpy as jnp
import numpy as np
from jax import lax
from jax.experimental import pallas as pl
from jax.experimental.pallas import tpu as pltpu
from jax.experimental.pallas import tpu_sc as plsc

K = 10
N = 10000
D = 64
DF = 128
E = 320000

NC, NS = 2, 16
NW = NC * NS
RPT = 640
NPAD = RPT * NS
EPW = 10240
EPAD = EPW * NW
CHUNK = 128
NCHUNK = EPW // CHUNK

ROW_BLK = 1280
GRID = NPAD // ROW_BLK


def _bern_matrix():
    M = np.zeros((K + 1, K + 1), dtype=np.float64)
    for i in range(K + 1):
        for j in range(K + 1):
            g = 0
            for m in range(0, i + 1):
                if 0 <= j - m <= K - i:
                    g += (-1) ** m * math.comb(i, m) * math.comb(K - i, j - m)
            M[j, i] = (math.comb(K, i) / (2 ** K)) * g
    return M


_BERN_M = _bern_matrix()

_SC_MESH = plsc.VectorSubcoreMesh(core_axis_name="c", subcore_axis_name="s")


NBUF = 4
DEGK = 8


def _sc_deg_body(row_hbm, ones_hbm, zeros1_hbm, deg_out, idx_all, ones_v, sem,
                 acc):
    cid = lax.axis_index("c")
    sid = lax.axis_index("s")
    wid = sid * NC + cid
    pltpu.sync_copy(zeros1_hbm.at[pl.ds(sid * RPT, RPT)],
                    acc.at[pl.ds(sid * RPT, RPT)])
    pltpu.sync_copy(ones_hbm, ones_v)
    pltpu.sync_copy(row_hbm.at[wid], idx_all)
    plsc.subcore_barrier()

    def outer(t, carry):
        for b in range(DEGK):
            pltpu.async_copy(ones_v, acc.at[idx_all.at[t * DEGK + b]], sem,
                             add=True)
        for b in range(DEGK):
            pltpu.make_async_copy(ones_v, acc.at[idx_all.at[t * DEGK + b]],
                                  sem).wait()
        return carry

    lax.fori_loop(0, NCHUNK // DEGK, outer, 0)
    plsc.subcore_barrier()
    pltpu.sync_copy(acc.at[pl.ds(sid * RPT, RPT)],
                    deg_out.at[cid, pl.ds(sid * RPT, RPT)])


@functools.partial(
    pl.kernel,
    mesh=_SC_MESH,
    compiler_params=pltpu.CompilerParams(use_tc_tiling_on_sc=False),
    out_type=jax.ShapeDtypeStruct((NC, NPAD), jnp.float32),
    scratch_types=[
        pltpu.VMEM((NCHUNK, CHUNK), jnp.int32),
        pltpu.VMEM((CHUNK,), jnp.float32),
        pltpu.SemaphoreType.DMA,
        pltpu.VMEM_SHARED((NPAD,), jnp.float32),
    ],
)
def _sc_degree(row_hbm, ones_hbm, zeros1_hbm, deg_out, idx_all, ones_v, sem,
               acc):
    _sc_deg_body(row_hbm, ones_hbm, zeros1_hbm, deg_out, idx_all, ones_v, sem,
                 acc)


PD = 2
NBUF2 = 4
HALF = NCHUNK // 2
NH = 20


def _sc_spmm_body(p_hbm, src_hbm, dst_hbm, zeros2_hbm, q_out,
                  idxs_all, idxd_all, rows, gsems, ssems, pstage, acc):
    cid = lax.axis_index("c")
    sid = lax.axis_index("s")
    wid = sid * NC + cid
    pltpu.sync_copy(zeros2_hbm.at[pl.ds(sid * RPT, RPT)],
                    acc.at[pl.ds(sid * RPT, RPT)])
    pltpu.sync_copy(p_hbm.at[pl.ds(sid * RPT, RPT)],
                    pstage.at[pl.ds(sid * RPT, RPT)])
    plsc.subcore_barrier()

    def g_cp(i, b):
        return pltpu.make_async_copy(pstage.at[idxs_all.at[i]], rows[b],
                                     gsems[b])

    def s_cp(i, b):
        return pltpu.make_async_copy(rows[b], acc.at[idxd_all.at[i]],
                                     ssems[b])

    for ph in range(2):
        pltpu.sync_copy(src_hbm.at[wid, pl.ds(ph * HALF, HALF)], idxs_all)
        pltpu.sync_copy(dst_hbm.at[wid, pl.ds(ph * HALF, HALF)], idxd_all)

        def issue_gather(j, b):
            if ph == 0:
                j = jnp.int32(j)

                @pl.when(j < NH)
                def _hbm():
                    pltpu.async_copy(p_hbm.at[idxs_all.at[j]], rows[b],
                                     gsems[b])

                @pl.when(j >= NH)
                def _spm():
                    pltpu.async_copy(pstage.at[idxs_all.at[j]], rows[b],
                                     gsems[b])
            else:
                pltpu.async_copy(pstage.at[idxs_all.at[j]], rows[b],
                                 gsems[b])

        for i in range(PD):
            issue_gather(i, i % NBUF2)

        def outer(t, carry):
            for b0 in range(NBUF2):
                j = t * NBUF2 + b0
                bp = (b0 + PD) % NBUF2

                @pl.when(j + PD < HALF)
                def _prefetch():
                    @pl.when(j + PD >= NBUF2)
                    def _free():
                        s_cp(j + PD - NBUF2, bp).wait()

                    issue_gather(j + PD, bp)

                g_cp(j, b0).wait()
                pltpu.async_copy(rows[b0], acc.at[idxd_all.at[j]], ssems[b0],
                                 add=True)
            return carry

        lax.fori_loop(0, HALF // NBUF2, outer, 0)
        for i in range(HALF - NBUF2, HALF):
            s_cp(i, i % NBUF2).wait()

    plsc.subcore_barrier()
    pltpu.sync_copy(acc.at[pl.ds(sid * RPT, RPT)],
                    q_out.at[cid, pl.ds(sid * RPT, RPT)])


@functools.partial(
    pl.kernel,
    mesh=_SC_MESH,
    compiler_params=pltpu.CompilerParams(use_tc_tiling_on_sc=False),
    out_type=jax.ShapeDtypeStruct((NC, NPAD, D), jnp.float32),
    scratch_types=[
        pltpu.VMEM((HALF, CHUNK), jnp.int32),
        pltpu.VMEM((HALF, CHUNK), jnp.int32),
        [pltpu.VMEM((CHUNK, D), jnp.float32) for _ in range(NBUF2)],
        [pltpu.SemaphoreType.DMA for _ in range(NBUF2)],
        [pltpu.SemaphoreType.DMA for _ in range(NBUF2)],
        pltpu.VMEM_SHARED((NPAD, D), jnp.float32),
        pltpu.VMEM_SHARED((NPAD, D), jnp.float32),
    ],
)
def _sc_spmm(p_hbm, src_hbm, dst_hbm, zeros2_hbm, q_out,
             idxs_all, idxd_all, rows, gsems, ssems, pstage, acc):
    _sc_spmm_body(p_hbm, src_hbm, dst_hbm, zeros2_hbm, q_out,
                  idxs_all, idxd_all, rows, gsems, ssems, pstage, acc)


SUBC = RPT // CHUNK


def _sc_step_body(qprev_hbm, d2b_hbm, src_hbm, dst_hbm, zeros2_hbm, q_out,
                  idxs_all, idxd_all, rows, gsems, ssems, pstage, acc):
    cid = lax.axis_index("c")
    sid = lax.axis_index("s")
    wid = sid * NC + cid
    pltpu.sync_copy(zeros2_hbm.at[pl.ds(sid * RPT, RPT)],
                    acc.at[pl.ds(sid * RPT, RPT)])

    for c in range(SUBC):
        rb = sid * RPT + c * CHUNK
        pltpu.async_copy(qprev_hbm.at[0, pl.ds(rb, CHUNK)], rows[0],
                         gsems[0])
        pltpu.async_copy(qprev_hbm.at[1, pl.ds(rb, CHUNK)], rows[1],
                         gsems[1])
        pltpu.async_copy(d2b_hbm.at[pl.ds(rb, CHUNK)], rows[2], gsems[2])
        pltpu.make_async_copy(qprev_hbm.at[0, pl.ds(rb, CHUNK)], rows[0],
                              gsems[0]).wait()
        pltpu.make_async_copy(qprev_hbm.at[1, pl.ds(rb, CHUNK)], rows[1],
                              gsems[1]).wait()
        pltpu.make_async_copy(d2b_hbm.at[pl.ds(rb, CHUNK)], rows[2],
                              gsems[2]).wait()

        def comb(i, carry):
            for u in range(2):
                r = i * 2 + u
                for c4 in range(D // 16):
                    sl = pl.ds(c4 * 16, 16)
                    v = rows[2][r, sl] * (rows[0][r, sl] + rows[1][r, sl])
                    rows[0][r, sl] = v
            return carry

        lax.fori_loop(0, CHUNK // 2, comb, 0)
        pltpu.sync_copy(rows[0], pstage.at[pl.ds(rb, CHUNK)])

    plsc.subcore_barrier()

    def g_cp(i, b):
        return pltpu.make_async_copy(pstage.at[idxs_all.at[i]], rows[b],
                                     gsems[b])

    def s_cp(i, b):
        return pltpu.make_async_copy(rows[b], acc.at[idxd_all.at[i]],
                                     ssems[b])

    for ph in range(2):
        pltpu.sync_copy(src_hbm.at[wid, pl.ds(ph * HALF, HALF)], idxs_all)
        pltpu.sync_copy(dst_hbm.at[wid, pl.ds(ph * HALF, HALF)], idxd_all)
        for i in range(PD):
            g_cp(i, i % NBUF2).start()

        def outer(t, carry):
            for b0 in range(NBUF2):
                j = t * NBUF2 + b0
                bp = (b0 + PD) % NBUF2

                @pl.when(j + PD < HALF)
                def _prefetch():
                    @pl.when(j + PD >= NBUF2)
                    def _free():
                        s_cp(j + PD - NBUF2, bp).wait()

                    pltpu.async_copy(pstage.at[idxs_all.at[j + PD]],
                                     rows[bp], gsems[bp])

                g_cp(j, b0).wait()
                pltpu.async_copy(rows[b0], acc.at[idxd_all.at[j]], ssems[b0],
                                 add=True)
            return carry

        lax.fori_loop(0, HALF // NBUF2, outer, 0)
        for i in range(HALF - NBUF2, HALF):
            s_cp(i, i % NBUF2).wait()

    plsc.subcore_barrier()
    pltpu.sync_copy(acc.at[pl.ds(sid * RPT, RPT)],
                    q_out.at[cid, pl.ds(sid * RPT, RPT)])


@functools.partial(
    pl.kernel,
    mesh=_SC_MESH,
    compiler_params=pltpu.CompilerParams(use_tc_tiling_on_sc=False),
    out_type=jax.ShapeDtypeStruct((NC, NPAD, D), jnp.float32),
    scratch_types=[
        pltpu.VMEM((HALF, CHUNK), jnp.int32),
        pltpu.VMEM((HALF, CHUNK), jnp.int32),
        [pltpu.VMEM((CHUNK, D), jnp.float32) for _ in range(NBUF2)],
        [pltpu.SemaphoreType.DMA for _ in range(NBUF2)],
        [pltpu.SemaphoreType.DMA for _ in range(NBUF2)],
        pltpu.VMEM_SHARED((NPAD, D), jnp.float32),
        pltpu.VMEM_SHARED((NPAD, D), jnp.float32),
    ],
)
def _sc_step(qprev_hbm, d2b_hbm, src_hbm, dst_hbm, zeros2_hbm, q_out,
             idxs_all, idxd_all, rows, gsems, ssems, pstage, acc):
    _sc_step_body(qprev_hbm, d2b_hbm, src_hbm, dst_hbm, zeros2_hbm, q_out,
                  idxs_all, idxd_all, rows, gsems, ssems, pstage, acc)


def _mlp_body(x_ref, w1t_ref, b1_ref, w2t_ref, b2_ref, o_ref):
    h = jnp.dot(x_ref[...], w1t_ref[...], preferred_element_type=jnp.float32)
    h = jnp.maximum(h + b1_ref[...], 0.0)
    o_ref[...] = (
        jnp.dot(h, w2t_ref[...], preferred_element_type=jnp.float32)
        + b2_ref[...]
    )


_mlp = pl.pallas_call(
    _mlp_body,
    grid=(GRID,),
    in_specs=[
        pl.BlockSpec((ROW_BLK, DF), lambda i: (i, 0)),
        pl.BlockSpec((DF, DF), lambda i: (0, 0)),
        pl.BlockSpec((1, DF), lambda i: (0, 0)),
        pl.BlockSpec((DF, D), lambda i: (0, 0)),
        pl.BlockSpec((1, D), lambda i: (0, 0)),
    ],
    out_specs=pl.BlockSpec((ROW_BLK, D), lambda i: (i, 0)),
    out_shape=jax.ShapeDtypeStruct((NPAD, D), jnp.float32),
)


def _prep_body(degp_ref, h_ref, d0_ref, dinvb_ref, d2b_ref, p_ref, acc_ref):
    deg = degp_ref[0, :] + degp_ref[1, :]
    dinv = jnp.where(deg > 0, lax.rsqrt(deg), 0.0)
    dinvb = jnp.broadcast_to(dinv[:, None], (ROW_BLK, D))
    h = h_ref[...]
    dinvb_ref[...] = dinvb
    d2b_ref[...] = dinvb * dinvb
    p_ref[...] = dinvb * h
    acc_ref[...] = d0_ref[0, 0] * h


_prep = pl.pallas_call(
    _prep_body,
    grid=(GRID,),
    in_specs=[
        pl.BlockSpec((2, ROW_BLK), lambda i: (0, i)),
        pl.BlockSpec((ROW_BLK, D), lambda i: (i, 0)),
        pl.BlockSpec((1, 1), lambda i: (0, 0)),
    ],
    out_specs=[
        pl.BlockSpec((ROW_BLK, D), lambda i: (i, 0)),
        pl.BlockSpec((ROW_BLK, D), lambda i: (i, 0)),
        pl.BlockSpec((ROW_BLK, D), lambda i: (i, 0)),
        pl.BlockSpec((ROW_BLK, D), lambda i: (i, 0)),
    ],
    out_shape=[
        jax.ShapeDtypeStruct((NPAD, D), jnp.float32),
        jax.ShapeDtypeStruct((NPAD, D), jnp.float32),
        jax.ShapeDtypeStruct((NPAD, D), jnp.float32),
        jax.ShapeDtypeStruct((NPAD, D), jnp.float32),
    ],
)


def _combine_body(q_ref, dinvb_ref, accin_ref, dj_ref, accout_ref):
    y = dinvb_ref[...] * (q_ref[0] + q_ref[1])
    accout_ref[...] = accin_ref[...] + dj_ref[0, 0] * y


_combine = pl.pallas_call(
    _combine_body,
    grid=(GRID,),
    in_specs=[
        pl.BlockSpec((2, ROW_BLK, D), lambda i: (0, i, 0)),
        pl.BlockSpec((ROW_BLK, D), lambda i: (i, 0)),
        pl.BlockSpec((ROW_BLK, D), lambda i: (i, 0)),
        pl.BlockSpec((1, 1), lambda i: (0, 0)),
    ],
    out_specs=pl.BlockSpec((ROW_BLK, D), lambda i: (i, 0)),
    out_shape=jax.ShapeDtypeStruct((NPAD, D), jnp.float32),
)


def _final_body(q_ref, dinvb_ref, accin_ref, dj_ref, out_ref):
    y = dinvb_ref[...] * (q_ref[0] + q_ref[1])
    acc = accin_ref[...] + dj_ref[0, 0] * y
    m = jnp.max(acc, axis=1, keepdims=True)
    lse = m + jnp.log(jnp.sum(jnp.exp(acc - m), axis=1, keepdims=True))
    out_ref[...] = acc - lse


_final = pl.pallas_call(
    _final_body,
    grid=(GRID,),
    in_specs=[
        pl.BlockSpec((2, ROW_BLK, D), lambda i: (0, i, 0)),
        pl.BlockSpec((ROW_BLK, D), lambda i: (i, 0)),
        pl.BlockSpec((ROW_BLK, D), lambda i: (i, 0)),
        pl.BlockSpec((1, 1), lambda i: (0, 0)),
    ],
    out_specs=pl.BlockSpec((ROW_BLK, D), lambda i: (i, 0)),
    out_shape=jax.ShapeDtypeStruct((NPAD, D), jnp.float32),
)


def kernel(x, edge_index, W1, b1, W2, b2, temp):
    row = edge_index[0].astype(jnp.int32)
    col = edge_index[1].astype(jnp.int32)
    pad = jnp.full((EPAD - E,), NPAD - 1, jnp.int32)
    src = jnp.concatenate([row, pad]).reshape(NW, NCHUNK, CHUNK)
    dst = jnp.concatenate([col, pad]).reshape(NW, NCHUNK, CHUNK)

    xpad = jnp.pad(x, ((0, NPAD - N), (0, 0)))
    ones_c = jnp.ones((CHUNK,), jnp.float32)
    zeros1 = jnp.zeros((NPAD,), jnp.float32)
    zeros2 = jnp.zeros((NPAD, D), jnp.float32)

    d = jnp.asarray(_BERN_M, jnp.float32) @ jax.nn.relu(temp)

    h = _mlp(xpad, W1.T, b1[None, :], W2.T, b2[None, :])
    degp = _sc_degree(src, ones_c, zeros1)
    dinvb, d2b, p, acc = _prep(degp, h, d[0].reshape(1, 1))
    q = _sc_spmm(p, src, dst, zeros2)
    for j in range(1, K):
        acc = _combine(q, dinvb, acc, d[j].reshape(1, 1))
        q = _sc_step(q, d2b, src, dst, zeros2)
    out = _final(q, dinvb, acc, d[K].reshape(1, 1))
    return out[:N]


# speedup vs baseline: 92.5786x; 1.0150x over previous
import functools
import math

import jax
import jax.numpy as jnp
import numpy as np
from jax import lax
from jax.experimental import pallas as pl
from jax.experimental.pallas import tpu as pltpu
from jax.experimental.pallas import tpu_sc as plsc

K = 10
N = 10000
D = 64
DF = 128
E = 320000

NC, NS = 2, 16
NW = NC * NS
RPT = 640
NPAD = RPT * NS
EPW = 10240
EPAD = EPW * NW
CHUNK = 128
NCHUNK = EPW // CHUNK

ROW_BLK = 1280
GRID = NPAD // ROW_BLK


def _bern_matrix():
    M = np.zeros((K + 1, K + 1), dtype=np.float64)
    for i in range(K + 1):
        for j in range(K + 1):
            g = 0
            for m in range(0, i + 1):
                if 0 <= j - m <= K - i:
                    g += (-1) ** m * math.comb(i, m) * math.comb(K - i, j - m)
            M[j, i] = (math.comb(K, i) / (2 ** K)) * g
    return M


_BERN_M = _bern_matrix()

_SC_MESH = plsc.VectorSubcoreMesh(core_axis_name="c", subcore_axis_name="s")


NBUF = 4
DEGK = 8


def _sc_deg_body(row_hbm, ones_hbm, zeros1_hbm, deg_out, idx_all, ones_v, sem,
                 acc):
    cid = lax.axis_index("c")
    sid = lax.axis_index("s")
    wid = sid * NC + cid
    pltpu.sync_copy(zeros1_hbm.at[pl.ds(sid * RPT, RPT)],
                    acc.at[pl.ds(sid * RPT, RPT)])
    pltpu.sync_copy(ones_hbm, ones_v)
    pltpu.sync_copy(row_hbm.at[wid], idx_all)
    plsc.subcore_barrier()

    def outer(t, carry):
        for b in range(DEGK):
            pltpu.async_copy(ones_v, acc.at[idx_all.at[t * DEGK + b]], sem,
                             add=True)
        for b in range(DEGK):
            pltpu.make_async_copy(ones_v, acc.at[idx_all.at[t * DEGK + b]],
                                  sem).wait()
        return carry

    lax.fori_loop(0, NCHUNK // DEGK, outer, 0)
    plsc.subcore_barrier()
    pltpu.sync_copy(acc.at[pl.ds(sid * RPT, RPT)],
                    deg_out.at[cid, pl.ds(sid * RPT, RPT)])


@functools.partial(
    pl.kernel,
    mesh=_SC_MESH,
    compiler_params=pltpu.CompilerParams(use_tc_tiling_on_sc=False),
    out_type=jax.ShapeDtypeStruct((NC, NPAD), jnp.float32),
    scratch_types=[
        pltpu.VMEM((NCHUNK, CHUNK), jnp.int32),
        pltpu.VMEM((CHUNK,), jnp.float32),
        pltpu.SemaphoreType.DMA,
        pltpu.VMEM_SHARED((NPAD,), jnp.float32),
    ],
)
def _sc_degree(row_hbm, ones_hbm, zeros1_hbm, deg_out, idx_all, ones_v, sem,
               acc):
    _sc_deg_body(row_hbm, ones_hbm, zeros1_hbm, deg_out, idx_all, ones_v, sem,
                 acc)


PD = 2
NBUF2 = 4
HALF = NCHUNK // 2
NH = 20


def _sc_spmm_body(p_hbm, src_hbm, dst_hbm, zeros2_hbm, q_out,
                  idxs_all, idxd_all, rows, gsems, ssems, pstage, acc):
    cid = lax.axis_index("c")
    sid = lax.axis_index("s")
    wid = sid * NC + cid
    zcp = pltpu.make_async_copy(zeros2_hbm.at[pl.ds(sid * RPT, RPT)],
                                acc.at[pl.ds(sid * RPT, RPT)], ssems[0])
    zcp.start()
    pcp = pltpu.make_async_copy(p_hbm.at[pl.ds(sid * RPT, RPT)],
                                pstage.at[pl.ds(sid * RPT, RPT)], ssems[1])
    pcp.start()
    pltpu.sync_copy(src_hbm.at[wid, pl.ds(0, HALF)], idxs_all)
    pltpu.sync_copy(dst_hbm.at[wid, pl.ds(0, HALF)], idxd_all)
    zcp.wait()
    pcp.wait()
    plsc.subcore_barrier()

    def g_cp(i, b):
        return pltpu.make_async_copy(pstage.at[idxs_all.at[i]], rows[b],
                                     gsems[b])

    def s_cp(i, b):
        return pltpu.make_async_copy(rows[b], acc.at[idxd_all.at[i]],
                                     ssems[b])

    for ph in range(2):
        if ph > 0:
            pltpu.sync_copy(src_hbm.at[wid, pl.ds(ph * HALF, HALF)],
                            idxs_all)
            pltpu.sync_copy(dst_hbm.at[wid, pl.ds(ph * HALF, HALF)],
                            idxd_all)

        def issue_gather(j, b):
            if ph == 0:
                j = jnp.int32(j)

                @pl.when(j < NH)
                def _hbm():
                    pltpu.async_copy(p_hbm.at[idxs_all.at[j]], rows[b],
                                     gsems[b])

                @pl.when(j >= NH)
                def _spm():
                    pltpu.async_copy(pstage.at[idxs_all.at[j]], rows[b],
                                     gsems[b])
            else:
                pltpu.async_copy(pstage.at[idxs_all.at[j]], rows[b],
                                 gsems[b])

        for i in range(PD):
            issue_gather(i, i % NBUF2)

        def outer(t, carry):
            for b0 in range(NBUF2):
                j = t * NBUF2 + b0
                bp = (b0 + PD) % NBUF2

                @pl.when(j + PD < HALF)
                def _prefetch():
                    @pl.when(j + PD >= NBUF2)
                    def _free():
                        s_cp(j + PD - NBUF2, bp).wait()

                    issue_gather(j + PD, bp)

                g_cp(j, b0).wait()
                pltpu.async_copy(rows[b0], acc.at[idxd_all.at[j]], ssems[b0],
                                 add=True)
            return carry

        lax.fori_loop(0, HALF // NBUF2, outer, 0)
        for i in range(HALF - NBUF2, HALF):
            s_cp(i, i % NBUF2).wait()

    plsc.subcore_barrier()
    pltpu.sync_copy(acc.at[pl.ds(sid * RPT, RPT)],
                    q_out.at[cid, pl.ds(sid * RPT, RPT)])


@functools.partial(
    pl.kernel,
    mesh=_SC_MESH,
    compiler_params=pltpu.CompilerParams(use_tc_tiling_on_sc=False),
    out_type=jax.ShapeDtypeStruct((NC, NPAD, D), jnp.float32),
    scratch_types=[
        pltpu.VMEM((HALF, CHUNK), jnp.int32),
        pltpu.VMEM((HALF, CHUNK), jnp.int32),
        [pltpu.VMEM((CHUNK, D), jnp.float32) for _ in range(NBUF2)],
        [pltpu.SemaphoreType.DMA for _ in range(NBUF2)],
        [pltpu.SemaphoreType.DMA for _ in range(NBUF2)],
        pltpu.VMEM_SHARED((NPAD, D), jnp.float32),
        pltpu.VMEM_SHARED((NPAD, D), jnp.float32),
    ],
)
def _sc_spmm(p_hbm, src_hbm, dst_hbm, zeros2_hbm, q_out,
             idxs_all, idxd_all, rows, gsems, ssems, pstage, acc):
    _sc_spmm_body(p_hbm, src_hbm, dst_hbm, zeros2_hbm, q_out,
                  idxs_all, idxd_all, rows, gsems, ssems, pstage, acc)


SUBC = RPT // CHUNK


def _sc_step_body(qprev_hbm, d2b_hbm, src_hbm, dst_hbm, zeros2_hbm, q_out,
                  idxs_all, idxd_all, rows, gsems, ssems, pstage, acc):
    cid = lax.axis_index("c")
    sid = lax.axis_index("s")
    wid = sid * NC + cid
    zcp = pltpu.make_async_copy(zeros2_hbm.at[pl.ds(sid * RPT, RPT)],
                                acc.at[pl.ds(sid * RPT, RPT)], ssems[0])
    zcp.start()

    for c in range(SUBC):
        rb = sid * RPT + c * CHUNK
        pltpu.async_copy(qprev_hbm.at[0, pl.ds(rb, CHUNK)], rows[0],
                         gsems[0])
        pltpu.async_copy(qprev_hbm.at[1, pl.ds(rb, CHUNK)], rows[1],
                         gsems[1])
        pltpu.async_copy(d2b_hbm.at[pl.ds(rb, CHUNK)], rows[2], gsems[2])
        pltpu.make_async_copy(qprev_hbm.at[0, pl.ds(rb, CHUNK)], rows[0],
                              gsems[0]).wait()
        pltpu.make_async_copy(qprev_hbm.at[1, pl.ds(rb, CHUNK)], rows[1],
                              gsems[1]).wait()
        pltpu.make_async_copy(d2b_hbm.at[pl.ds(rb, CHUNK)], rows[2],
                              gsems[2]).wait()

        def comb(i, carry):
            for u in range(4):
                r = i * 4 + u
                for c4 in range(D // 16):
                    sl = pl.ds(c4 * 16, 16)
                    v = rows[2][r, sl] * (rows[0][r, sl] + rows[1][r, sl])
                    rows[0][r, sl] = v
            return carry

        lax.fori_loop(0, CHUNK // 4, comb, 0)
        pltpu.sync_copy(rows[0], pstage.at[pl.ds(rb, CHUNK)])

    pltpu.sync_copy(src_hbm.at[wid, pl.ds(0, HALF)], idxs_all)
    pltpu.sync_copy(dst_hbm.at[wid, pl.ds(0, HALF)], idxd_all)
    zcp.wait()
    plsc.subcore_barrier()

    def g_cp(i, b):
        return pltpu.make_async_copy(pstage.at[idxs_all.at[i]], rows[b],
                                     gsems[b])

    def s_cp(i, b):
        return pltpu.make_async_copy(rows[b], acc.at[idxd_all.at[i]],
                                     ssems[b])

    for ph in range(2):
        if ph > 0:
            pltpu.sync_copy(src_hbm.at[wid, pl.ds(ph * HALF, HALF)],
                            idxs_all)
            pltpu.sync_copy(dst_hbm.at[wid, pl.ds(ph * HALF, HALF)],
                            idxd_all)
        for i in range(PD):
            g_cp(i, i % NBUF2).start()

        def outer(t, carry):
            for b0 in range(NBUF2):
                j = t * NBUF2 + b0
                bp = (b0 + PD) % NBUF2

                @pl.when(j + PD < HALF)
                def _prefetch():
                    @pl.when(j + PD >= NBUF2)
                    def _free():
                        s_cp(j + PD - NBUF2, bp).wait()

                    pltpu.async_copy(pstage.at[idxs_all.at[j + PD]],
                                     rows[bp], gsems[bp])

                g_cp(j, b0).wait()
                pltpu.async_copy(rows[b0], acc.at[idxd_all.at[j]], ssems[b0],
                                 add=True)
            return carry

        lax.fori_loop(0, HALF // NBUF2, outer, 0)
        for i in range(HALF - NBUF2, HALF):
            s_cp(i, i % NBUF2).wait()

    plsc.subcore_barrier()
    pltpu.sync_copy(acc.at[pl.ds(sid * RPT, RPT)],
                    q_out.at[cid, pl.ds(sid * RPT, RPT)])


@functools.partial(
    pl.kernel,
    mesh=_SC_MESH,
    compiler_params=pltpu.CompilerParams(use_tc_tiling_on_sc=False),
    out_type=jax.ShapeDtypeStruct((NC, NPAD, D), jnp.float32),
    scratch_types=[
        pltpu.VMEM((HALF, CHUNK), jnp.int32),
        pltpu.VMEM((HALF, CHUNK), jnp.int32),
        [pltpu.VMEM((CHUNK, D), jnp.float32) for _ in range(NBUF2)],
        [pltpu.SemaphoreType.DMA for _ in range(NBUF2)],
        [pltpu.SemaphoreType.DMA for _ in range(NBUF2)],
        pltpu.VMEM_SHARED((NPAD, D), jnp.float32),
        pltpu.VMEM_SHARED((NPAD, D), jnp.float32),
    ],
)
def _sc_step(qprev_hbm, d2b_hbm, src_hbm, dst_hbm, zeros2_hbm, q_out,
             idxs_all, idxd_all, rows, gsems, ssems, pstage, acc):
    _sc_step_body(qprev_hbm, d2b_hbm, src_hbm, dst_hbm, zeros2_hbm, q_out,
                  idxs_all, idxd_all, rows, gsems, ssems, pstage, acc)


def _mlp_body(x_ref, w1t_ref, b1_ref, w2t_ref, b2_ref, o_ref):
    h = jnp.dot(x_ref[...], w1t_ref[...], preferred_element_type=jnp.float32)
    h = jnp.maximum(h + b1_ref[...], 0.0)
    o_ref[...] = (
        jnp.dot(h, w2t_ref[...], preferred_element_type=jnp.float32)
        + b2_ref[...]
    )


_mlp = pl.pallas_call(
    _mlp_body,
    grid=(GRID,),
    in_specs=[
        pl.BlockSpec((ROW_BLK, DF), lambda i: (i, 0)),
        pl.BlockSpec((DF, DF), lambda i: (0, 0)),
        pl.BlockSpec((1, DF), lambda i: (0, 0)),
        pl.BlockSpec((DF, D), lambda i: (0, 0)),
        pl.BlockSpec((1, D), lambda i: (0, 0)),
    ],
    out_specs=pl.BlockSpec((ROW_BLK, D), lambda i: (i, 0)),
    out_shape=jax.ShapeDtypeStruct((NPAD, D), jnp.float32),
)


def _prep_body(degp_ref, h_ref, d0_ref, dinvb_ref, d2b_ref, p_ref, acc_ref):
    deg = degp_ref[0, :] + degp_ref[1, :]
    dinv = jnp.where(deg > 0, lax.rsqrt(deg), 0.0)
    dinvb = jnp.broadcast_to(dinv[:, None], (ROW_BLK, D))
    h = h_ref[...]
    dinvb_ref[...] = dinvb
    d2b_ref[...] = dinvb * dinvb
    p_ref[...] = dinvb * h
    acc_ref[...] = d0_ref[0, 0] * h


_prep = pl.pallas_call(
    _prep_body,
    grid=(GRID,),
    in_specs=[
        pl.BlockSpec((2, ROW_BLK), lambda i: (0, i)),
        pl.BlockSpec((ROW_BLK, D), lambda i: (i, 0)),
        pl.BlockSpec((1, 1), lambda i: (0, 0)),
    ],
    out_specs=[
        pl.BlockSpec((ROW_BLK, D), lambda i: (i, 0)),
        pl.BlockSpec((ROW_BLK, D), lambda i: (i, 0)),
        pl.BlockSpec((ROW_BLK, D), lambda i: (i, 0)),
        pl.BlockSpec((ROW_BLK, D), lambda i: (i, 0)),
    ],
    out_shape=[
        jax.ShapeDtypeStruct((NPAD, D), jnp.float32),
        jax.ShapeDtypeStruct((NPAD, D), jnp.float32),
        jax.ShapeDtypeStruct((NPAD, D), jnp.float32),
        jax.ShapeDtypeStruct((NPAD, D), jnp.float32),
    ],
)


def _combine_body(q_ref, dinvb_ref, accin_ref, dj_ref, accout_ref):
    y = dinvb_ref[...] * (q_ref[0] + q_ref[1])
    accout_ref[...] = accin_ref[...] + dj_ref[0, 0] * y


_combine = pl.pallas_call(
    _combine_body,
    grid=(GRID,),
    in_specs=[
        pl.BlockSpec((2, ROW_BLK, D), lambda i: (0, i, 0)),
        pl.BlockSpec((ROW_BLK, D), lambda i: (i, 0)),
        pl.BlockSpec((ROW_BLK, D), lambda i: (i, 0)),
        pl.BlockSpec((1, 1), lambda i: (0, 0)),
    ],
    out_specs=pl.BlockSpec((ROW_BLK, D), lambda i: (i, 0)),
    out_shape=jax.ShapeDtypeStruct((NPAD, D), jnp.float32),
)


def _final_body(q_ref, dinvb_ref, accin_ref, dj_ref, out_ref):
    y = dinvb_ref[...] * (q_ref[0] + q_ref[1])
    acc = accin_ref[...] + dj_ref[0, 0] * y
    m = jnp.max(acc, axis=1, keepdims=True)
    lse = m + jnp.log(jnp.sum(jnp.exp(acc - m), axis=1, keepdims=True))
    out_ref[...] = acc - lse


_final = pl.pallas_call(
    _final_body,
    grid=(GRID,),
    in_specs=[
        pl.BlockSpec((2, ROW_BLK, D), lambda i: (0, i, 0)),
        pl.BlockSpec((ROW_BLK, D), lambda i: (i, 0)),
        pl.BlockSpec((ROW_BLK, D), lambda i: (i, 0)),
        pl.BlockSpec((1, 1), lambda i: (0, 0)),
    ],
    out_specs=pl.BlockSpec((ROW_BLK, D), lambda i: (i, 0)),
    out_shape=jax.ShapeDtypeStruct((NPAD, D), jnp.float32),
)


def kernel(x, edge_index, W1, b1, W2, b2, temp):
    row = edge_index[0].astype(jnp.int32)
    col = edge_index[1].astype(jnp.int32)
    pad = jnp.full((EPAD - E,), NPAD - 1, jnp.int32)
    src = jnp.concatenate([row, pad]).reshape(NW, NCHUNK, CHUNK)
    dst = jnp.concatenate([col, pad]).reshape(NW, NCHUNK, CHUNK)

    xpad = jnp.pad(x, ((0, NPAD - N), (0, 0)))
    ones_c = jnp.ones((CHUNK,), jnp.float32)
    zeros1 = jnp.zeros((NPAD,), jnp.float32)
    zeros2 = jnp.zeros((NPAD, D), jnp.float32)

    d = jnp.asarray(_BERN_M, jnp.float32) @ jax.nn.relu(temp)

    h = _mlp(xpad, W1.T, b1[None, :], W2.T, b2[None, :])
    degp = _sc_degree(src, ones_c, zeros1)
    dinvb, d2b, p, acc = _prep(degp, h, d[0].reshape(1, 1))
    q = _sc_spmm(p, src, dst, zeros2)
    for j in range(1, K):
        acc = _combine(q, dinvb, acc, d[j].reshape(1, 1))
        q = _sc_step(q, d2b, src, dst, zeros2)
    out = _final(q, dinvb, acc, d[K].reshape(1, 1))
    return out[:N]


# speedup vs baseline: 93.9415x; 1.0147x over previous
import functools
import math

import jax
import jax.numpy as jnp
import numpy as np
from jax import lax
from jax.experimental import pallas as pl
from jax.experimental.pallas import tpu as pltpu
from jax.experimental.pallas import tpu_sc as plsc

K = 10
N = 10000
D = 64
DF = 128
E = 320000

NC, NS = 2, 16
NW = NC * NS
RPT = 640
NPAD = RPT * NS
EPW = 10240
EPAD = EPW * NW
CHUNK = 128
NCHUNK = EPW // CHUNK

ROW_BLK = 1280
GRID = NPAD // ROW_BLK


def _bern_matrix():
    M = np.zeros((K + 1, K + 1), dtype=np.float64)
    for i in range(K + 1):
        for j in range(K + 1):
            g = 0
            for m in range(0, i + 1):
                if 0 <= j - m <= K - i:
                    g += (-1) ** m * math.comb(i, m) * math.comb(K - i, j - m)
            M[j, i] = (math.comb(K, i) / (2 ** K)) * g
    return M


_BERN_M = _bern_matrix()

_SC_MESH = plsc.VectorSubcoreMesh(core_axis_name="c", subcore_axis_name="s")


NBUF = 4
DEGK = 8


def _sc_deg_body(row_hbm, ones_hbm, zeros1_hbm, deg_out, idx_all, ones_v, sem,
                 acc):
    cid = lax.axis_index("c")
    sid = lax.axis_index("s")
    wid = sid * NC + cid
    pltpu.sync_copy(zeros1_hbm.at[pl.ds(sid * RPT, RPT)],
                    acc.at[pl.ds(sid * RPT, RPT)])
    pltpu.sync_copy(ones_hbm, ones_v)
    pltpu.sync_copy(row_hbm.at[wid], idx_all)
    plsc.subcore_barrier()

    def outer(t, carry):
        for b in range(DEGK):
            pltpu.async_copy(ones_v, acc.at[idx_all.at[t * DEGK + b]], sem,
                             add=True)
        for b in range(DEGK):
            pltpu.make_async_copy(ones_v, acc.at[idx_all.at[t * DEGK + b]],
                                  sem).wait()
        return carry

    lax.fori_loop(0, NCHUNK // DEGK, outer, 0)
    plsc.subcore_barrier()
    pltpu.sync_copy(acc.at[pl.ds(sid * RPT, RPT)],
                    deg_out.at[cid, pl.ds(sid * RPT, RPT)])


@functools.partial(
    pl.kernel,
    mesh=_SC_MESH,
    compiler_params=pltpu.CompilerParams(use_tc_tiling_on_sc=False),
    out_type=jax.ShapeDtypeStruct((NC, NPAD), jnp.float32),
    scratch_types=[
        pltpu.VMEM((NCHUNK, CHUNK), jnp.int32),
        pltpu.VMEM((CHUNK,), jnp.float32),
        pltpu.SemaphoreType.DMA,
        pltpu.VMEM_SHARED((NPAD,), jnp.float32),
    ],
)
def _sc_degree(row_hbm, ones_hbm, zeros1_hbm, deg_out, idx_all, ones_v, sem,
               acc):
    _sc_deg_body(row_hbm, ones_hbm, zeros1_hbm, deg_out, idx_all, ones_v, sem,
                 acc)


PD = 2
NBUF2 = 4
HALF = NCHUNK // 2
NH = 20


def _sc_spmm_body(p_hbm, src_hbm, dst_hbm, zeros2_hbm, q_out,
                  idxs_all, idxd_all, rows, gsems, ssems, pstage, acc):
    cid = lax.axis_index("c")
    sid = lax.axis_index("s")
    wid = sid * NC + cid
    zcp = pltpu.make_async_copy(zeros2_hbm.at[pl.ds(sid * RPT, RPT)],
                                acc.at[pl.ds(sid * RPT, RPT)], ssems[0])
    zcp.start()
    pcp = pltpu.make_async_copy(p_hbm.at[pl.ds(sid * RPT, RPT)],
                                pstage.at[pl.ds(sid * RPT, RPT)], ssems[1])
    pcp.start()
    pltpu.sync_copy(src_hbm.at[wid, pl.ds(0, HALF)], idxs_all)
    pltpu.sync_copy(dst_hbm.at[wid, pl.ds(0, HALF)], idxd_all)
    zcp.wait()
    pcp.wait()
    plsc.subcore_barrier()

    def g_cp(i, b):
        return pltpu.make_async_copy(pstage.at[idxs_all.at[i]], rows[b],
                                     gsems[b])

    def s_cp(i, b):
        return pltpu.make_async_copy(rows[b], acc.at[idxd_all.at[i]],
                                     ssems[b])

    for ph in range(2):
        if ph > 0:
            pltpu.sync_copy(src_hbm.at[wid, pl.ds(ph * HALF, HALF)],
                            idxs_all)
            pltpu.sync_copy(dst_hbm.at[wid, pl.ds(ph * HALF, HALF)],
                            idxd_all)

        def issue_gather(j, b):
            if ph == 0:
                j = jnp.int32(j)

                @pl.when(j < NH)
                def _hbm():
                    pltpu.async_copy(p_hbm.at[idxs_all.at[j]], rows[b],
                                     gsems[b])

                @pl.when(j >= NH)
                def _spm():
                    pltpu.async_copy(pstage.at[idxs_all.at[j]], rows[b],
                                     gsems[b])
            else:
                pltpu.async_copy(pstage.at[idxs_all.at[j]], rows[b],
                                 gsems[b])

        for i in range(PD):
            issue_gather(i, i % NBUF2)

        def outer(t, carry):
            for b0 in range(NBUF2):
                j = t * NBUF2 + b0
                bp = (b0 + PD) % NBUF2

                @pl.when(j + PD < HALF)
                def _prefetch():
                    @pl.when(j + PD >= NBUF2)
                    def _free():
                        s_cp(j + PD - NBUF2, bp).wait()

                    issue_gather(j + PD, bp)

                g_cp(j, b0).wait()
                pltpu.async_copy(rows[b0], acc.at[idxd_all.at[j]], ssems[b0],
                                 add=True)
            return carry

        lax.fori_loop(0, HALF // NBUF2, outer, 0)
        for i in range(HALF - NBUF2, HALF):
            s_cp(i, i % NBUF2).wait()

    plsc.subcore_barrier()
    pltpu.sync_copy(acc.at[pl.ds(sid * RPT, RPT)],
                    q_out.at[cid, pl.ds(sid * RPT, RPT)])


@functools.partial(
    pl.kernel,
    mesh=_SC_MESH,
    compiler_params=pltpu.CompilerParams(use_tc_tiling_on_sc=False),
    out_type=jax.ShapeDtypeStruct((NC, NPAD, D), jnp.float32),
    scratch_types=[
        pltpu.VMEM((HALF, CHUNK), jnp.int32),
        pltpu.VMEM((HALF, CHUNK), jnp.int32),
        [pltpu.VMEM((CHUNK, D), jnp.float32) for _ in range(NBUF2)],
        [pltpu.SemaphoreType.DMA for _ in range(NBUF2)],
        [pltpu.SemaphoreType.DMA for _ in range(NBUF2)],
        pltpu.VMEM_SHARED((NPAD, D), jnp.float32),
        pltpu.VMEM_SHARED((NPAD, D), jnp.float32),
    ],
)
def _sc_spmm(p_hbm, src_hbm, dst_hbm, zeros2_hbm, q_out,
             idxs_all, idxd_all, rows, gsems, ssems, pstage, acc):
    _sc_spmm_body(p_hbm, src_hbm, dst_hbm, zeros2_hbm, q_out,
                  idxs_all, idxd_all, rows, gsems, ssems, pstage, acc)


SUBC = RPT // CHUNK


def _sc_step_body(qprev_hbm, d2b_hbm, src_hbm, dst_hbm, zeros2_hbm, q_out,
                  idxs_all, idxd_all, rows, gsems, ssems, zsem, pstage, acc):
    cid = lax.axis_index("c")
    sid = lax.axis_index("s")
    wid = sid * NC + cid
    zcp = pltpu.make_async_copy(zeros2_hbm.at[pl.ds(sid * RPT, RPT)],
                                acc.at[pl.ds(sid * RPT, RPT)], zsem)
    zcp.start()

    HROW = CHUNK // 2
    NSUB = RPT // HROW

    def a_cps(c):
        par = c % 2
        off = par * HROW
        rb = sid * RPT + c * HROW
        return (
            pltpu.make_async_copy(qprev_hbm.at[0, pl.ds(rb, HROW)],
                                  rows[0].at[pl.ds(off, HROW)], gsems[par]),
            pltpu.make_async_copy(qprev_hbm.at[1, pl.ds(rb, HROW)],
                                  rows[1].at[pl.ds(off, HROW)],
                                  gsems[2 + par]),
            pltpu.make_async_copy(d2b_hbm.at[pl.ds(rb, HROW)],
                                  rows[2].at[pl.ds(off, HROW)], ssems[par]),
        )

    def p_cp(c):
        par = c % 2
        off = par * HROW
        rb = sid * RPT + c * HROW
        return pltpu.make_async_copy(rows[3].at[pl.ds(off, HROW)],
                                     pstage.at[pl.ds(rb, HROW)],
                                     ssems[2 + par])

    for cp in a_cps(0):
        cp.start()
    for cp in a_cps(1):
        cp.start()
    for c in range(NSUB):
        par = c % 2
        off = par * HROW
        for cp in a_cps(c):
            cp.wait()
        if c >= 2:
            p_cp(c - 2).wait()

        def comb(i, carry):
            for u in range(4):
                r = off + i * 4 + u
                for c4 in range(D // 16):
                    sl = pl.ds(c4 * 16, 16)
                    v = rows[2][r, sl] * (rows[0][r, sl] + rows[1][r, sl])
                    rows[3][r, sl] = v
            return carry

        lax.fori_loop(0, HROW // 4, comb, 0)
        p_cp(c).start()
        if c + 2 < NSUB:
            for cp in a_cps(c + 2):
                cp.start()
    p_cp(NSUB - 2).wait()
    p_cp(NSUB - 1).wait()

    pltpu.sync_copy(src_hbm.at[wid, pl.ds(0, HALF)], idxs_all)
    pltpu.sync_copy(dst_hbm.at[wid, pl.ds(0, HALF)], idxd_all)
    zcp.wait()
    plsc.subcore_barrier()

    def g_cp(i, b):
        return pltpu.make_async_copy(pstage.at[idxs_all.at[i]], rows[b],
                                     gsems[b])

    def s_cp(i, b):
        return pltpu.make_async_copy(rows[b], acc.at[idxd_all.at[i]],
                                     ssems[b])

    for ph in range(2):
        if ph > 0:
            pltpu.sync_copy(src_hbm.at[wid, pl.ds(ph * HALF, HALF)],
                            idxs_all)
            pltpu.sync_copy(dst_hbm.at[wid, pl.ds(ph * HALF, HALF)],
                            idxd_all)
        for i in range(PD):
            g_cp(i, i % NBUF2).start()

        def outer(t, carry):
            for b0 in range(NBUF2):
                j = t * NBUF2 + b0
                bp = (b0 + PD) % NBUF2

                @pl.when(j + PD < HALF)
                def _prefetch():
                    @pl.when(j + PD >= NBUF2)
                    def _free():
                        s_cp(j + PD - NBUF2, bp).wait()

                    pltpu.async_copy(pstage.at[idxs_all.at[j + PD]],
                                     rows[bp], gsems[bp])

                g_cp(j, b0).wait()
                pltpu.async_copy(rows[b0], acc.at[idxd_all.at[j]], ssems[b0],
                                 add=True)
            return carry

        lax.fori_loop(0, HALF // NBUF2, outer, 0)
        for i in range(HALF - NBUF2, HALF):
            s_cp(i, i % NBUF2).wait()

    plsc.subcore_barrier()
    pltpu.sync_copy(acc.at[pl.ds(sid * RPT, RPT)],
                    q_out.at[cid, pl.ds(sid * RPT, RPT)])


@functools.partial(
    pl.kernel,
    mesh=_SC_MESH,
    compiler_params=pltpu.CompilerParams(use_tc_tiling_on_sc=False),
    out_type=jax.ShapeDtypeStruct((NC, NPAD, D), jnp.float32),
    scratch_types=[
        pltpu.VMEM((HALF, CHUNK), jnp.int32),
        pltpu.VMEM((HALF, CHUNK), jnp.int32),
        [pltpu.VMEM((CHUNK, D), jnp.float32) for _ in range(NBUF2)],
        [pltpu.SemaphoreType.DMA for _ in range(NBUF2)],
        [pltpu.SemaphoreType.DMA for _ in range(NBUF2)],
        pltpu.SemaphoreType.DMA,
        pltpu.VMEM_SHARED((NPAD, D), jnp.float32),
        pltpu.VMEM_SHARED((NPAD, D), jnp.float32),
    ],
)
def _sc_step(qprev_hbm, d2b_hbm, src_hbm, dst_hbm, zeros2_hbm, q_out,
             idxs_all, idxd_all, rows, gsems, ssems, zsem, pstage, acc):
    _sc_step_body(qprev_hbm, d2b_hbm, src_hbm, dst_hbm, zeros2_hbm, q_out,
                  idxs_all, idxd_all, rows, gsems, ssems, zsem, pstage, acc)


def _mlp_body(x_ref, w1t_ref, b1_ref, w2t_ref, b2_ref, o_ref):
    h = jnp.dot(x_ref[...], w1t_ref[...], preferred_element_type=jnp.float32)
    h = jnp.maximum(h + b1_ref[...], 0.0)
    o_ref[...] = (
        jnp.dot(h, w2t_ref[...], preferred_element_type=jnp.float32)
        + b2_ref[...]
    )


_mlp = pl.pallas_call(
    _mlp_body,
    grid=(GRID,),
    in_specs=[
        pl.BlockSpec((ROW_BLK, DF), lambda i: (i, 0)),
        pl.BlockSpec((DF, DF), lambda i: (0, 0)),
        pl.BlockSpec((1, DF), lambda i: (0, 0)),
        pl.BlockSpec((DF, D), lambda i: (0, 0)),
        pl.BlockSpec((1, D), lambda i: (0, 0)),
    ],
    out_specs=pl.BlockSpec((ROW_BLK, D), lambda i: (i, 0)),
    out_shape=jax.ShapeDtypeStruct((NPAD, D), jnp.float32),
)


def _prep_body(degp_ref, h_ref, d0_ref, dinvb_ref, d2b_ref, p_ref, acc_ref):
    deg = degp_ref[0, :] + degp_ref[1, :]
    dinv = jnp.where(deg > 0, lax.rsqrt(deg), 0.0)
    dinvb = jnp.broadcast_to(dinv[:, None], (ROW_BLK, D))
    h = h_ref[...]
    dinvb_ref[...] = dinvb
    d2b_ref[...] = dinvb * dinvb
    p_ref[...] = dinvb * h
    acc_ref[...] = d0_ref[0, 0] * h


_prep = pl.pallas_call(
    _prep_body,
    grid=(GRID,),
    in_specs=[
        pl.BlockSpec((2, ROW_BLK), lambda i: (0, i)),
        pl.BlockSpec((ROW_BLK, D), lambda i: (i, 0)),
        pl.BlockSpec((1, 1), lambda i: (0, 0)),
    ],
    out_specs=[
        pl.BlockSpec((ROW_BLK, D), lambda i: (i, 0)),
        pl.BlockSpec((ROW_BLK, D), lambda i: (i, 0)),
        pl.BlockSpec((ROW_BLK, D), lambda i: (i, 0)),
        pl.BlockSpec((ROW_BLK, D), lambda i: (i, 0)),
    ],
    out_shape=[
        jax.ShapeDtypeStruct((NPAD, D), jnp.float32),
        jax.ShapeDtypeStruct((NPAD, D), jnp.float32),
        jax.ShapeDtypeStruct((NPAD, D), jnp.float32),
        jax.ShapeDtypeStruct((NPAD, D), jnp.float32),
    ],
)


def _combine_body(q_ref, dinvb_ref, accin_ref, dj_ref, accout_ref):
    y = dinvb_ref[...] * (q_ref[0] + q_ref[1])
    accout_ref[...] = accin_ref[...] + dj_ref[0, 0] * y


_combine = pl.pallas_call(
    _combine_body,
    grid=(GRID,),
    in_specs=[
        pl.BlockSpec((2, ROW_BLK, D), lambda i: (0, i, 0)),
        pl.BlockSpec((ROW_BLK, D), lambda i: (i, 0)),
        pl.BlockSpec((ROW_BLK, D), lambda i: (i, 0)),
        pl.BlockSpec((1, 1), lambda i: (0, 0)),
    ],
    out_specs=pl.BlockSpec((ROW_BLK, D), lambda i: (i, 0)),
    out_shape=jax.ShapeDtypeStruct((NPAD, D), jnp.float32),
)


def _final_body(q_ref, dinvb_ref, accin_ref, dj_ref, out_ref):
    y = dinvb_ref[...] * (q_ref[0] + q_ref[1])
    acc = accin_ref[...] + dj_ref[0, 0] * y
    m = jnp.max(acc, axis=1, keepdims=True)
    lse = m + jnp.log(jnp.sum(jnp.exp(acc - m), axis=1, keepdims=True))
    out_ref[...] = acc - lse


_final = pl.pallas_call(
    _final_body,
    grid=(GRID,),
    in_specs=[
        pl.BlockSpec((2, ROW_BLK, D), lambda i: (0, i, 0)),
        pl.BlockSpec((ROW_BLK, D), lambda i: (i, 0)),
        pl.BlockSpec((ROW_BLK, D), lambda i: (i, 0)),
        pl.BlockSpec((1, 1), lambda i: (0, 0)),
    ],
    out_specs=pl.BlockSpec((ROW_BLK, D), lambda i: (i, 0)),
    out_shape=jax.ShapeDtypeStruct((NPAD, D), jnp.float32),
)


def kernel(x, edge_index, W1, b1, W2, b2, temp):
    row = edge_index[0].astype(jnp.int32)
    col = edge_index[1].astype(jnp.int32)
    pad = jnp.full((EPAD - E,), NPAD - 1, jnp.int32)
    src = jnp.concatenate([row, pad]).reshape(NW, NCHUNK, CHUNK)
    dst = jnp.concatenate([col, pad]).reshape(NW, NCHUNK, CHUNK)

    xpad = jnp.pad(x, ((0, NPAD - N), (0, 0)))
    ones_c = jnp.ones((CHUNK,), jnp.float32)
    zeros1 = jnp.zeros((NPAD,), jnp.float32)
    zeros2 = jnp.zeros((NPAD, D), jnp.float32)

    d = jnp.asarray(_BERN_M, jnp.float32) @ jax.nn.relu(temp)

    h = _mlp(xpad, W1.T, b1[None, :], W2.T, b2[None, :])
    degp = _sc_degree(src, ones_c, zeros1)
    dinvb, d2b, p, acc = _prep(degp, h, d[0].reshape(1, 1))
    q = _sc_spmm(p, src, dst, zeros2)
    for j in range(1, K):
        acc = _combine(q, dinvb, acc, d[j].reshape(1, 1))
        q = _sc_step(q, d2b, src, dst, zeros2)
    out = _final(q, dinvb, acc, d[K].reshape(1, 1))
    return out[:N]
